# Initial kernel scaffold; baseline (speedup 1.0000x reference)
import jax
import jax.numpy as jnp
from jax.experimental import pallas as pl


def kernel(nodes, senders, receivers, eps, w_enc, b_enc, w_fc, b_fc, w_mean, b_mean, w_logstd, b_logstd, w_dec, b_dec, w_out, b_out):
    raise NotImplementedError("write your pallas kernel here")


# speedup vs baseline: 4.5772x; 4.5772x over previous
import functools

import jax
import jax.numpy as jnp
from jax import lax
from jax.experimental import pallas as pl
from jax.experimental.pallas import tpu as pltpu
from jax.experimental.pallas import tpu_sc as plsc

B = 50
N = 200
HG = 32
HFC = 256
LAT = 64
OUT = 128
DFEAT = 128
E = 160000
NN = B * N
NP = 10112
RPT = 632
NW = 32
LW = 128
CH = 40
EP = NW * CH * LW
GB = 16
RB = NP // GB

_SLABS = ((0, 128), (128, 128), (256, 128), (384, 128), (512, 120))

_mesh = plsc.VectorSubcoreMesh(core_axis_name="c", subcore_axis_name="s")
_sc_params = pltpu.CompilerParams(use_tc_tiling_on_sc=False)


def _sc_degrees(s_p, r_p):
    out_t = (jax.ShapeDtypeStruct((2, NP, 16), jnp.float32),
             jax.ShapeDtypeStruct((2, NP, 16), jnp.float32))

    @functools.partial(
        pl.kernel, mesh=_mesh, out_type=out_t, compiler_params=_sc_params,
        scratch_types=[
            pltpu.VMEM((CH, LW), jnp.int32),
            pltpu.VMEM((CH, LW), jnp.int32),
            pltpu.VMEM((LW, 16), jnp.float32),
            pltpu.VMEM((RPT, 16), jnp.float32),
            pltpu.VMEM_SHARED((NP, 16), jnp.float32),
            pltpu.VMEM_SHARED((NP, 16), jnp.float32),
        ])
    def deg_kernel(s_hbm, r_hbm, ds_out, dr_out, sidx_v, ridx_v, ones_v,
                   zero_v, ds_sh, dr_sh):
        c = lax.axis_index("c")
        s = lax.axis_index("s")
        wid = s * 2 + c
        pltpu.sync_copy(s_hbm.at[wid], sidx_v)
        pltpu.sync_copy(r_hbm.at[wid], ridx_v)

        @pl.loop(0, LW)
        def _(i):
            ones_v[i, :] = jnp.ones((16,), jnp.float32)

        @pl.loop(0, RPT)
        def _(i):
            zero_v[i, :] = jnp.zeros((16,), jnp.float32)

        base = s * RPT
        pltpu.sync_copy(zero_v, ds_sh.at[pl.ds(base, RPT)])
        pltpu.sync_copy(zero_v, dr_sh.at[pl.ds(base, RPT)])
        plsc.subcore_barrier()

        @pl.loop(0, CH)
        def _(i):
            pltpu.sync_copy(ones_v, ds_sh.at[sidx_v.at[i]], add=True)
            pltpu.sync_copy(ones_v, dr_sh.at[ridx_v.at[i]], add=True)

        plsc.subcore_barrier()
        pltpu.sync_copy(ds_sh.at[pl.ds(base, RPT)],
                        ds_out.at[c, pl.ds(base, RPT)])
        pltpu.sync_copy(dr_sh.at[pl.ds(base, RPT)],
                        dr_out.at[c, pl.ds(base, RPT)])

    return deg_kernel(s_p, r_p)


def _sc_segsum(table, s_p, r_p, d):
    @functools.partial(
        pl.kernel, mesh=_mesh,
        out_type=jax.ShapeDtypeStruct((2, NP, d), jnp.float32),
        compiler_params=_sc_params,
        scratch_types=[
            pltpu.VMEM((CH, LW), jnp.int32),
            pltpu.VMEM((CH, LW), jnp.int32),
            pltpu.VMEM((LW, d), jnp.float32),
            pltpu.VMEM_SHARED((NP, d), jnp.float32),
        ])
    def seg_kernel(tab_hbm, s_hbm, r_hbm, out_hbm, sidx_v, ridx_v, rows_v,
                   acc_sh):
        c = lax.axis_index("c")
        s = lax.axis_index("s")
        wid = s * 2 + c
        pltpu.sync_copy(s_hbm.at[wid], sidx_v)
        pltpu.sync_copy(r_hbm.at[wid], ridx_v)

        @pl.loop(0, LW)
        def _(i):
            @pl.loop(0, d, step=16)
            def _(l):
                rows_v[i, pl.ds(l, 16)] = jnp.zeros((16,), jnp.float32)

        base = s * RPT
        for off, sz in _SLABS:
            pltpu.sync_copy(rows_v.at[pl.ds(0, sz)],
                            acc_sh.at[pl.ds(base + off, sz)])
        plsc.subcore_barrier()

        @pl.loop(0, CH)
        def _(i):
            pltpu.sync_copy(tab_hbm.at[sidx_v.at[i]], rows_v)
            pltpu.sync_copy(rows_v, acc_sh.at[ridx_v.at[i]], add=True)

        plsc.subcore_barrier()
        for off, sz in _SLABS:
            pltpu.sync_copy(acc_sh.at[pl.ds(base + off, sz)],
                            out_hbm.at[c, pl.ds(base + off, sz)])

    return seg_kernel(table, s_p, r_p)


def _row_spec(width):
    return pl.BlockSpec((RB, width), lambda i: (i, 0))


def _full_spec(shape):
    return pl.BlockSpec(shape, lambda i: tuple(0 for _ in shape))


def _tc_encoder(nodes_p, w_enc, b_enc, ds0, ds1):
    def body(n_ref, w_ref, b_ref, ds0_ref, ds1_ref, hn_ref):
        h = jnp.dot(n_ref[...], w_ref[...],
                    preferred_element_type=jnp.float32,
                    precision=lax.Precision.HIGHEST)
        h = jnp.maximum(h + b_ref[...], 0.0)
        ds = ds0_ref[...] + ds1_ref[...]
        is1 = lax.rsqrt(ds + 1.0)
        hn_ref[...] = h * jnp.concatenate([is1, is1], axis=1)

    return pl.pallas_call(
        body,
        grid=(GB,),
        in_specs=[_row_spec(DFEAT), _full_spec((DFEAT, HG)),
                  _full_spec((1, HG)), _row_spec(16), _row_spec(16)],
        out_specs=_row_spec(HG),
        out_shape=jax.ShapeDtypeStruct((NP, HG), jnp.float32),
    )(nodes_p, w_enc, b_enc, ds0, ds1)


def _tc_combine1(p0, p1, hn, dr0, dr1):
    def body(p0_ref, p1_ref, hn_ref, dr0_ref, dr1_ref, o_ref):
        dr = dr0_ref[...] + dr1_ref[...]
        ir1 = lax.rsqrt(dr + 1.0)
        o_ref[...] = (p0_ref[...] + p1_ref[...] + hn_ref[...]) * \
            jnp.concatenate([ir1, ir1], axis=1)

    return pl.pallas_call(
        body,
        grid=(GB,),
        in_specs=[_row_spec(HG), _row_spec(HG), _row_spec(HG),
                  _row_spec(16), _row_spec(16)],
        out_specs=_row_spec(HG),
        out_shape=jax.ShapeDtypeStruct((NP, HG), jnp.float32),
    )(p0, p1, hn, dr0, dr1)


def _tc_middle(x, w_fc, b_fc, w_mean, b_mean, w_logstd, b_logstd, eps,
               w_dec, b_dec):
    def body(x_ref, wfc_ref, bfc_ref, wm_ref, bm_ref, wl_ref, bl_ref,
             eps_ref, wd_ref, bd_ref, mean_ref, logstd_ref, zh_ref):
        x = jnp.dot(x_ref[...], wfc_ref[...],
                    preferred_element_type=jnp.float32,
                    precision=lax.Precision.HIGHEST)
        x = jnp.maximum(x + bfc_ref[...], 0.0)
        mean = jnp.dot(x, wm_ref[...], preferred_element_type=jnp.float32,
                       precision=lax.Precision.HIGHEST) + bm_ref[...]
        logstd = jnp.dot(x, wl_ref[...], preferred_element_type=jnp.float32,
                         precision=lax.Precision.HIGHEST) + bl_ref[...]
        z = mean + jnp.exp(logstd) * eps_ref[...]
        zh = jnp.dot(z, wd_ref[...], preferred_element_type=jnp.float32,
                     precision=lax.Precision.HIGHEST)
        zh_ref[...] = jnp.maximum(zh + bd_ref[...], 0.0)
        mean_ref[...] = mean
        logstd_ref[...] = logstd

    return pl.pallas_call(
        body,
        out_shape=(jax.ShapeDtypeStruct((B, LAT), jnp.float32),
                   jax.ShapeDtypeStruct((B, LAT), jnp.float32),
                   jax.ShapeDtypeStruct((B, N * HG), jnp.float32)),
    )(x, w_fc, b_fc, w_mean, b_mean, w_logstd, b_logstd, eps, w_dec, b_dec)


def _tc_decoder_nodes(zh3, w_out, b_out, ds0, ds1):
    def body(z_ref, w_ref, b_ref, ds0_ref, ds1_ref, o_ref):
        z2 = jnp.dot(z_ref[...], w_ref[...],
                     preferred_element_type=jnp.float32,
                     precision=lax.Precision.HIGHEST) + b_ref[...]
        ds = ds0_ref[...] + ds1_ref[...]
        inv = lax.rsqrt(jnp.maximum(ds[:, :1], 1.0))
        o_ref[...] = z2 * jnp.broadcast_to(inv, (RB, OUT))

    return pl.pallas_call(
        body,
        grid=(GB,),
        in_specs=[_row_spec(HG), _full_spec((HG, OUT)), _full_spec((1, OUT)),
                  _row_spec(16), _row_spec(16)],
        out_specs=_row_spec(OUT),
        out_shape=jax.ShapeDtypeStruct((NP, OUT), jnp.float32),
    )(zh3, w_out, b_out, ds0, ds1)


def _tc_final(q0, q1, dr0, dr1):
    def body(q0_ref, q1_ref, dr0_ref, dr1_ref, o_ref):
        dr = dr0_ref[...] + dr1_ref[...]
        inv = lax.rsqrt(jnp.maximum(dr[:, :1], 1.0))
        o_ref[...] = (q0_ref[...] + q1_ref[...]) * jnp.broadcast_to(
            inv, (RB, OUT))

    return pl.pallas_call(
        body,
        grid=(GB,),
        in_specs=[_row_spec(OUT), _row_spec(OUT), _row_spec(16),
                  _row_spec(16)],
        out_specs=_row_spec(OUT),
        out_shape=jax.ShapeDtypeStruct((NP, OUT), jnp.float32),
    )(q0, q1, dr0, dr1)


def kernel(nodes, senders, receivers, eps, w_enc, b_enc, w_fc, b_fc, w_mean,
           b_mean, w_logstd, b_logstd, w_dec, b_dec, w_out, b_out):
    pad = jnp.full((EP - E,), NN, dtype=jnp.int32)
    s_p = jnp.concatenate([senders, pad]).reshape(NW, CH, LW)
    r_p = jnp.concatenate([receivers, pad]).reshape(NW, CH, LW)

    degs_p, degr_p = _sc_degrees(s_p, r_p)

    nodes_p = jnp.pad(nodes, ((0, NP - NN), (0, 0)))
    hnorm = _tc_encoder(nodes_p, w_enc, b_enc.reshape(1, HG),
                        degs_p[0], degs_p[1])

    seg1 = _sc_segsum(hnorm, s_p, r_p, HG)

    x1 = _tc_combine1(seg1[0], seg1[1], hnorm, degr_p[0], degr_p[1])
    x = x1[:NN].reshape(B, N * HG)

    mean, log_std, zh = _tc_middle(
        x, w_fc, b_fc.reshape(1, HFC), w_mean, b_mean.reshape(1, LAT),
        w_logstd, b_logstd.reshape(1, LAT), eps, w_dec,
        b_dec.reshape(1, N * HG))

    zh3 = jnp.pad(zh.reshape(NN, HG), ((0, NP - NN), (0, 0)))
    z2norm = _tc_decoder_nodes(zh3, w_out, b_out.reshape(1, OUT),
                               degs_p[0], degs_p[1])

    seg2 = _sc_segsum(z2norm, s_p, r_p, OUT)

    outp = _tc_final(seg2[0], seg2[1], degr_p[0], degr_p[1])
    return mean, log_std, outp[:NN]


# speedup vs baseline: 7.7777x; 1.6992x over previous
import functools

import jax
import jax.numpy as jnp
from jax import lax
from jax.experimental import pallas as pl
from jax.experimental.pallas import tpu as pltpu
from jax.experimental.pallas import tpu_sc as plsc

B = 50
N = 200
HG = 32
HFC = 256
LAT = 64
OUT = 128
DFEAT = 128
E = 160000
NN = B * N
NP = 10112
RPT = 632
NW = 32
LW = 128
CH = 40
EP = NW * CH * LW
EWR = E // NW
PADW = CH * LW - EWR
GB = 16
RB = NP // GB

_SLABS = ((0, 128), (128, 128), (256, 128), (384, 128), (512, 120))

_mesh = plsc.VectorSubcoreMesh(core_axis_name="c", subcore_axis_name="s")
_sc_params = pltpu.CompilerParams(use_tc_tiling_on_sc=False)


def _sc_degrees(s_p, r_p):
    out_t = (jax.ShapeDtypeStruct((2, NP, 16), jnp.float32),
             jax.ShapeDtypeStruct((2, NP, 16), jnp.float32))

    @functools.partial(
        pl.kernel, mesh=_mesh, out_type=out_t, compiler_params=_sc_params,
        scratch_types=[
            pltpu.VMEM((CH, LW), jnp.int32),
            pltpu.VMEM((CH, LW), jnp.int32),
            pltpu.VMEM((LW, 16), jnp.float32),
            pltpu.VMEM((RPT, 16), jnp.float32),
            pltpu.VMEM_SHARED((NP, 16), jnp.float32),
            pltpu.VMEM_SHARED((NP, 16), jnp.float32),
        ])
    def deg_kernel(s_hbm, r_hbm, ds_out, dr_out, sidx_v, ridx_v, ones_v,
                   zero_v, ds_sh, dr_sh):
        c = lax.axis_index("c")
        s = lax.axis_index("s")
        wid = s * 2 + c
        pltpu.sync_copy(s_hbm.at[wid], sidx_v)
        pltpu.sync_copy(r_hbm.at[wid], ridx_v)

        @pl.loop(0, LW)
        def _(i):
            ones_v[i, :] = jnp.ones((16,), jnp.float32)

        @pl.loop(0, RPT)
        def _(i):
            zero_v[i, :] = jnp.zeros((16,), jnp.float32)

        base = s * RPT
        pltpu.sync_copy(zero_v, ds_sh.at[pl.ds(base, RPT)])
        pltpu.sync_copy(zero_v, dr_sh.at[pl.ds(base, RPT)])
        plsc.subcore_barrier()

        @pl.loop(0, CH)
        def _(i):
            pltpu.sync_copy(ones_v, ds_sh.at[sidx_v.at[i]], add=True)
            pltpu.sync_copy(ones_v, dr_sh.at[ridx_v.at[i]], add=True)

        plsc.subcore_barrier()
        pltpu.sync_copy(ds_sh.at[pl.ds(base, RPT)],
                        ds_out.at[c, pl.ds(base, RPT)])
        pltpu.sync_copy(dr_sh.at[pl.ds(base, RPT)],
                        dr_out.at[c, pl.ds(base, RPT)])

    return deg_kernel(s_p, r_p)


def _sc_segsum(table, s_p, r_p, d):
    @functools.partial(
        pl.kernel, mesh=_mesh,
        out_type=jax.ShapeDtypeStruct((2, NP, d), jnp.float32),
        compiler_params=_sc_params,
        scratch_types=[
            pltpu.VMEM((CH, LW), jnp.int32),
            pltpu.VMEM((CH, LW), jnp.int32),
            pltpu.VMEM((LW, d), jnp.float32),
            pltpu.VMEM_SHARED((NP, d), jnp.float32),
        ])
    def seg_kernel(tab_hbm, s_hbm, r_hbm, out_hbm, sidx_v, ridx_v, rows_v,
                   acc_sh):
        c = lax.axis_index("c")
        s = lax.axis_index("s")
        wid = s * 2 + c
        pltpu.sync_copy(s_hbm.at[wid], sidx_v)
        pltpu.sync_copy(r_hbm.at[wid], ridx_v)

        @pl.loop(0, LW)
        def _(i):
            @pl.loop(0, d, step=16)
            def _(l):
                rows_v[i, pl.ds(l, 16)] = jnp.zeros((16,), jnp.float32)

        base = s * RPT
        for off, sz in _SLABS:
            pltpu.sync_copy(rows_v.at[pl.ds(0, sz)],
                            acc_sh.at[pl.ds(base + off, sz)])
        plsc.subcore_barrier()

        @pl.loop(0, CH)
        def _(i):
            pltpu.sync_copy(tab_hbm.at[sidx_v.at[i]], rows_v)
            pltpu.sync_copy(rows_v, acc_sh.at[ridx_v.at[i]], add=True)

        plsc.subcore_barrier()
        for off, sz in _SLABS:
            pltpu.sync_copy(acc_sh.at[pl.ds(base + off, sz)],
                            out_hbm.at[c, pl.ds(base + off, sz)])

    return seg_kernel(table, s_p, r_p)


def _row_spec(width):
    return pl.BlockSpec((RB, width), lambda i: (i, 0))


def _full_spec(shape):
    return pl.BlockSpec(shape, lambda i: tuple(0 for _ in shape))


def _tc_encoder(nodes_p, w_enc, b_enc, ds0, ds1):
    def body(n_ref, w_ref, b_ref, ds0_ref, ds1_ref, hn_ref):
        h = jnp.dot(n_ref[...], w_ref[...],
                    preferred_element_type=jnp.float32,
                    precision=lax.Precision.HIGHEST)
        h = jnp.maximum(h + b_ref[...], 0.0)
        ds = ds0_ref[...] + ds1_ref[...]
        is1 = lax.rsqrt(ds + 1.0)
        hn_ref[...] = h * jnp.concatenate([is1, is1], axis=1)

    return pl.pallas_call(
        body,
        grid=(GB,),
        in_specs=[_row_spec(DFEAT), _full_spec((DFEAT, HG)),
                  _full_spec((1, HG)), _row_spec(16), _row_spec(16)],
        out_specs=_row_spec(HG),
        out_shape=jax.ShapeDtypeStruct((NP, HG), jnp.float32),
    )(nodes_p, w_enc, b_enc, ds0, ds1)


def _tc_combine1(p0, p1, hn, dr0, dr1):
    def body(p0_ref, p1_ref, hn_ref, dr0_ref, dr1_ref, o_ref):
        dr = dr0_ref[...] + dr1_ref[...]
        ir1 = lax.rsqrt(dr + 1.0)
        o_ref[...] = (p0_ref[...] + p1_ref[...] + hn_ref[...]) * \
            jnp.concatenate([ir1, ir1], axis=1)

    return pl.pallas_call(
        body,
        grid=(GB,),
        in_specs=[_row_spec(HG), _row_spec(HG), _row_spec(HG),
                  _row_spec(16), _row_spec(16)],
        out_specs=_row_spec(HG),
        out_shape=jax.ShapeDtypeStruct((NP, HG), jnp.float32),
    )(p0, p1, hn, dr0, dr1)


def _tc_middle(x, w_fc, b_fc, w_mean, b_mean, w_logstd, b_logstd, eps,
               w_dec, b_dec):
    def body(x_ref, wfc_ref, bfc_ref, wm_ref, bm_ref, wl_ref, bl_ref,
             eps_ref, wd_ref, bd_ref, mean_ref, logstd_ref, zh_ref):
        x = jnp.dot(x_ref[...], wfc_ref[...],
                    preferred_element_type=jnp.float32,
                    precision=lax.Precision.HIGHEST)
        x = jnp.maximum(x + bfc_ref[...], 0.0)
        mean = jnp.dot(x, wm_ref[...], preferred_element_type=jnp.float32,
                       precision=lax.Precision.HIGHEST) + bm_ref[...]
        logstd = jnp.dot(x, wl_ref[...], preferred_element_type=jnp.float32,
                         precision=lax.Precision.HIGHEST) + bl_ref[...]
        z = mean + jnp.exp(logstd) * eps_ref[...]
        zh = jnp.dot(z, wd_ref[...], preferred_element_type=jnp.float32,
                     precision=lax.Precision.HIGHEST)
        zh_ref[...] = jnp.maximum(zh + bd_ref[...], 0.0)
        mean_ref[...] = mean
        logstd_ref[...] = logstd

    return pl.pallas_call(
        body,
        out_shape=(jax.ShapeDtypeStruct((B, LAT), jnp.float32),
                   jax.ShapeDtypeStruct((B, LAT), jnp.float32),
                   jax.ShapeDtypeStruct((B, N * HG), jnp.float32)),
    )(x, w_fc, b_fc, w_mean, b_mean, w_logstd, b_logstd, eps, w_dec, b_dec)


def _tc_u_table(zh3, ds0, ds1):
    def body(z_ref, ds0_ref, ds1_ref, o_ref):
        ds = ds0_ref[...] + ds1_ref[...]
        iv = lax.rsqrt(jnp.maximum(ds, 1.0))
        iv32 = jnp.concatenate([iv, iv], axis=1)
        o_ref[...] = jnp.concatenate([z_ref[...] * iv32, iv], axis=1)

    return pl.pallas_call(
        body,
        grid=(GB,),
        in_specs=[_row_spec(HG), _row_spec(16), _row_spec(16)],
        out_specs=_row_spec(48),
        out_shape=jax.ShapeDtypeStruct((NP, 48), jnp.float32),
    )(zh3, ds0, ds1)


def _tc_final(q0, q1, w_out, b_out, dr0, dr1):
    def body(q0_ref, q1_ref, w_ref, b_ref, dr0_ref, dr1_ref, o_ref):
        t = q0_ref[...] + q1_ref[...]
        u = lax.slice(t, (0, 0), (RB, HG))
        v = lax.slice(t, (0, HG), (RB, HG + 1))
        z2 = jnp.dot(u, w_ref[...], preferred_element_type=jnp.float32,
                     precision=lax.Precision.HIGHEST) + v * b_ref[...]
        dr = dr0_ref[...] + dr1_ref[...]
        inv = lax.rsqrt(jnp.maximum(dr[:, :1], 1.0))
        o_ref[...] = z2 * jnp.broadcast_to(inv, (RB, OUT))

    return pl.pallas_call(
        body,
        grid=(GB,),
        in_specs=[_row_spec(48), _row_spec(48), _full_spec((HG, OUT)),
                  _full_spec((1, OUT)), _row_spec(16), _row_spec(16)],
        out_specs=_row_spec(OUT),
        out_shape=jax.ShapeDtypeStruct((NP, OUT), jnp.float32),
    )(q0, q1, w_out, b_out, dr0, dr1)


def kernel(nodes, senders, receivers, eps, w_enc, b_enc, w_fc, b_fc, w_mean,
           b_mean, w_logstd, b_logstd, w_dec, b_dec, w_out, b_out):
    padv = NN + (jnp.arange(PADW, dtype=jnp.int32) % (NP - NN))
    padb = jnp.broadcast_to(padv[None, :], (NW, PADW))
    s_p = jnp.concatenate(
        [senders.reshape(NW, EWR), padb], axis=1).reshape(NW, CH, LW)
    r_p = jnp.concatenate(
        [receivers.reshape(NW, EWR), padb], axis=1).reshape(NW, CH, LW)

    degs_p, degr_p = _sc_degrees(s_p, r_p)

    nodes_p = jnp.pad(nodes, ((0, NP - NN), (0, 0)))
    hnorm = _tc_encoder(nodes_p, w_enc, b_enc.reshape(1, HG),
                        degs_p[0], degs_p[1])

    seg1 = _sc_segsum(hnorm, s_p, r_p, HG)

    x1 = _tc_combine1(seg1[0], seg1[1], hnorm, degr_p[0], degr_p[1])
    x = x1[:NN].reshape(B, N * HG)

    mean, log_std, zh = _tc_middle(
        x, w_fc, b_fc.reshape(1, HFC), w_mean, b_mean.reshape(1, LAT),
        w_logstd, b_logstd.reshape(1, LAT), eps, w_dec,
        b_dec.reshape(1, N * HG))

    zh3 = jnp.pad(zh.reshape(NN, HG), ((0, NP - NN), (0, 0)))
    u48 = _tc_u_table(zh3, degs_p[0], degs_p[1])

    seg2 = _sc_segsum(u48, s_p, r_p, 48)

    outp = _tc_final(seg2[0], seg2[1], w_out, b_out.reshape(1, OUT),
                     degr_p[0], degr_p[1])
    return mean, log_std, outp[:NN]


# speedup vs baseline: 8.3223x; 1.0700x over previous
import functools

import jax
import jax.numpy as jnp
from jax import lax
from jax.experimental import pallas as pl
from jax.experimental.pallas import tpu as pltpu
from jax.experimental.pallas import tpu_sc as plsc

B = 50
N = 200
HG = 32
HFC = 256
LAT = 64
OUT = 128
DFEAT = 128
E = 160000
NN = B * N
NP = 10112
RPT = 632
NW = 32
LW = 128
CH = 40
EP = NW * CH * LW
EWR = E // NW
PADW = CH * LW - EWR
GB = 16
RB = NP // GB

_SLABS = ((0, 128), (128, 128), (256, 128), (384, 128), (512, 120))

_mesh = plsc.VectorSubcoreMesh(core_axis_name="c", subcore_axis_name="s")
_sc_params = pltpu.CompilerParams(use_tc_tiling_on_sc=False)


def _sc_degrees(s_p, r_p):
    out_t = (jax.ShapeDtypeStruct((2, NP, 16), jnp.float32),
             jax.ShapeDtypeStruct((2, NP, 16), jnp.float32))

    @functools.partial(
        pl.kernel, mesh=_mesh, out_type=out_t, compiler_params=_sc_params,
        scratch_types=[
            pltpu.VMEM((CH, LW), jnp.int32),
            pltpu.VMEM((CH, LW), jnp.int32),
            pltpu.VMEM((LW, 16), jnp.float32),
            pltpu.VMEM((RPT, 16), jnp.float32),
            pltpu.VMEM_SHARED((NP, 16), jnp.float32),
            pltpu.VMEM_SHARED((NP, 16), jnp.float32),
            pltpu.SemaphoreType.DMA,
            pltpu.SemaphoreType.DMA,
        ])
    def deg_kernel(s_hbm, r_hbm, ds_out, dr_out, sidx_v, ridx_v, ones_v,
                   zero_v, ds_sh, dr_sh, dsem, rsem):
        c = lax.axis_index("c")
        s = lax.axis_index("s")
        wid = s * 2 + c
        pltpu.sync_copy(s_hbm.at[wid], sidx_v)
        pltpu.sync_copy(r_hbm.at[wid], ridx_v)

        @pl.loop(0, LW)
        def _(i):
            ones_v[i, :] = jnp.ones((16,), jnp.float32)

        @pl.loop(0, RPT)
        def _(i):
            zero_v[i, :] = jnp.zeros((16,), jnp.float32)

        base = s * RPT
        pltpu.sync_copy(zero_v, ds_sh.at[pl.ds(base, RPT)])
        pltpu.sync_copy(zero_v, dr_sh.at[pl.ds(base, RPT)])
        plsc.subcore_barrier()

        @pl.loop(0, CH)
        def _(i):
            a = pltpu.async_copy(ones_v, ds_sh.at[sidx_v.at[i]], dsem,
                                 add=True)
            b = pltpu.async_copy(ones_v, dr_sh.at[ridx_v.at[i]], rsem,
                                 add=True)
            a.wait()
            b.wait()

        plsc.subcore_barrier()
        pltpu.sync_copy(ds_sh.at[pl.ds(base, RPT)],
                        ds_out.at[c, pl.ds(base, RPT)])
        pltpu.sync_copy(dr_sh.at[pl.ds(base, RPT)],
                        dr_out.at[c, pl.ds(base, RPT)])

    return deg_kernel(s_p, r_p)


def _sc_segsum(table, s_p, r_p, d):
    @functools.partial(
        pl.kernel, mesh=_mesh,
        out_type=jax.ShapeDtypeStruct((2, NP, d), jnp.float32),
        compiler_params=_sc_params,
        scratch_types=[
            pltpu.VMEM((CH, LW), jnp.int32),
            pltpu.VMEM((CH, LW), jnp.int32),
            pltpu.VMEM((LW, d), jnp.float32),
            pltpu.VMEM((LW, d), jnp.float32),
            pltpu.VMEM_SHARED((NP, d), jnp.float32),
            pltpu.SemaphoreType.DMA,
            pltpu.SemaphoreType.DMA,
        ])
    def seg_kernel(tab_hbm, s_hbm, r_hbm, out_hbm, sidx_v, ridx_v, rows0_v,
                   rows1_v, acc_sh, sem0, sem1):
        c = lax.axis_index("c")
        s = lax.axis_index("s")
        wid = s * 2 + c
        pltpu.sync_copy(s_hbm.at[wid], sidx_v)
        pltpu.sync_copy(r_hbm.at[wid], ridx_v)

        @pl.loop(0, LW)
        def _(i):
            @pl.loop(0, d, step=16)
            def _(l):
                rows0_v[i, pl.ds(l, 16)] = jnp.zeros((16,), jnp.float32)

        base = s * RPT
        for off, sz in _SLABS:
            pltpu.sync_copy(rows0_v.at[pl.ds(0, sz)],
                            acc_sh.at[pl.ds(base + off, sz)])
        plsc.subcore_barrier()

        pltpu.async_copy(tab_hbm.at[sidx_v.at[0]], rows0_v, sem0)

        @pl.loop(0, CH // 2)
        def _(j):
            i = j * 2
            pltpu.make_async_copy(tab_hbm.at[sidx_v.at[i]], rows0_v,
                                  sem0).wait()
            pltpu.async_copy(tab_hbm.at[sidx_v.at[i + 1]], rows1_v, sem1)
            pltpu.sync_copy(rows0_v, acc_sh.at[ridx_v.at[i]], add=True)
            pltpu.make_async_copy(tab_hbm.at[sidx_v.at[i + 1]], rows1_v,
                                  sem1).wait()

            @pl.when(j < CH // 2 - 1)
            def _():
                pltpu.async_copy(tab_hbm.at[sidx_v.at[i + 2]], rows0_v, sem0)

            pltpu.sync_copy(rows1_v, acc_sh.at[ridx_v.at[i + 1]], add=True)

        plsc.subcore_barrier()
        for off, sz in _SLABS:
            pltpu.sync_copy(acc_sh.at[pl.ds(base + off, sz)],
                            out_hbm.at[c, pl.ds(base + off, sz)])

    return seg_kernel(table, s_p, r_p)


def _row_spec(width):
    return pl.BlockSpec((RB, width), lambda i: (i, 0))


def _full_spec(shape):
    return pl.BlockSpec(shape, lambda i: tuple(0 for _ in shape))


def _tc_encoder(nodes_p, w_enc, b_enc, ds0, ds1):
    def body(n_ref, w_ref, b_ref, ds0_ref, ds1_ref, hn_ref):
        h = jnp.dot(n_ref[...], w_ref[...],
                    preferred_element_type=jnp.float32,
                    precision=lax.Precision.HIGHEST)
        h = jnp.maximum(h + b_ref[...], 0.0)
        ds = ds0_ref[...] + ds1_ref[...]
        is1 = lax.rsqrt(ds + 1.0)
        hn_ref[...] = h * jnp.concatenate([is1, is1], axis=1)

    return pl.pallas_call(
        body,
        grid=(GB,),
        in_specs=[_row_spec(DFEAT), _full_spec((DFEAT, HG)),
                  _full_spec((1, HG)), _row_spec(16), _row_spec(16)],
        out_specs=_row_spec(HG),
        out_shape=jax.ShapeDtypeStruct((NP, HG), jnp.float32),
    )(nodes_p, w_enc, b_enc, ds0, ds1)


def _tc_combine1(p0, p1, hn, dr0, dr1):
    def body(p0_ref, p1_ref, hn_ref, dr0_ref, dr1_ref, o_ref):
        dr = dr0_ref[...] + dr1_ref[...]
        ir1 = lax.rsqrt(dr + 1.0)
        o_ref[...] = (p0_ref[...] + p1_ref[...] + hn_ref[...]) * \
            jnp.concatenate([ir1, ir1], axis=1)

    return pl.pallas_call(
        body,
        grid=(GB,),
        in_specs=[_row_spec(HG), _row_spec(HG), _row_spec(HG),
                  _row_spec(16), _row_spec(16)],
        out_specs=_row_spec(HG),
        out_shape=jax.ShapeDtypeStruct((NP, HG), jnp.float32),
    )(p0, p1, hn, dr0, dr1)


def _tc_middle(x, w_fc, b_fc, w_mean, b_mean, w_logstd, b_logstd, eps,
               w_dec, b_dec):
    def body(x_ref, wfc_ref, bfc_ref, wm_ref, bm_ref, wl_ref, bl_ref,
             eps_ref, wd_ref, bd_ref, mean_ref, logstd_ref, zh_ref):
        x = jnp.dot(x_ref[...], wfc_ref[...],
                    preferred_element_type=jnp.float32,
                    precision=lax.Precision.HIGHEST)
        x = jnp.maximum(x + bfc_ref[...], 0.0)
        mean = jnp.dot(x, wm_ref[...], preferred_element_type=jnp.float32,
                       precision=lax.Precision.HIGHEST) + bm_ref[...]
        logstd = jnp.dot(x, wl_ref[...], preferred_element_type=jnp.float32,
                         precision=lax.Precision.HIGHEST) + bl_ref[...]
        z = mean + jnp.exp(logstd) * eps_ref[...]
        zh = jnp.dot(z, wd_ref[...], preferred_element_type=jnp.float32,
                     precision=lax.Precision.HIGHEST)
        zh_ref[...] = jnp.maximum(zh + bd_ref[...], 0.0)
        mean_ref[...] = mean
        logstd_ref[...] = logstd

    return pl.pallas_call(
        body,
        out_shape=(jax.ShapeDtypeStruct((B, LAT), jnp.float32),
                   jax.ShapeDtypeStruct((B, LAT), jnp.float32),
                   jax.ShapeDtypeStruct((B, N * HG), jnp.float32)),
    )(x, w_fc, b_fc, w_mean, b_mean, w_logstd, b_logstd, eps, w_dec, b_dec)


def _tc_u_table(zh3, ds0, ds1):
    def body(z_ref, ds0_ref, ds1_ref, o_ref):
        ds = ds0_ref[...] + ds1_ref[...]
        iv = lax.rsqrt(jnp.maximum(ds, 1.0))
        iv32 = jnp.concatenate([iv, iv], axis=1)
        o_ref[...] = jnp.concatenate([z_ref[...] * iv32, iv], axis=1)

    return pl.pallas_call(
        body,
        grid=(GB,),
        in_specs=[_row_spec(HG), _row_spec(16), _row_spec(16)],
        out_specs=_row_spec(48),
        out_shape=jax.ShapeDtypeStruct((NP, 48), jnp.float32),
    )(zh3, ds0, ds1)


def _tc_final(q0, q1, w_out, b_out, dr0, dr1):
    def body(q0_ref, q1_ref, w_ref, b_ref, dr0_ref, dr1_ref, o_ref):
        t = q0_ref[...] + q1_ref[...]
        u = lax.slice(t, (0, 0), (RB, HG))
        v = lax.slice(t, (0, HG), (RB, HG + 1))
        z2 = jnp.dot(u, w_ref[...], preferred_element_type=jnp.float32,
                     precision=lax.Precision.HIGHEST) + v * b_ref[...]
        dr = dr0_ref[...] + dr1_ref[...]
        inv = lax.rsqrt(jnp.maximum(dr[:, :1], 1.0))
        o_ref[...] = z2 * jnp.broadcast_to(inv, (RB, OUT))

    return pl.pallas_call(
        body,
        grid=(GB,),
        in_specs=[_row_spec(48), _row_spec(48), _full_spec((HG, OUT)),
                  _full_spec((1, OUT)), _row_spec(16), _row_spec(16)],
        out_specs=_row_spec(OUT),
        out_shape=jax.ShapeDtypeStruct((NP, OUT), jnp.float32),
    )(q0, q1, w_out, b_out, dr0, dr1)


def kernel(nodes, senders, receivers, eps, w_enc, b_enc, w_fc, b_fc, w_mean,
           b_mean, w_logstd, b_logstd, w_dec, b_dec, w_out, b_out):
    padv = NN + (jnp.arange(PADW, dtype=jnp.int32) % (NP - NN))
    padb = jnp.broadcast_to(padv[None, :], (NW, PADW))
    s_p = jnp.concatenate(
        [senders.reshape(NW, EWR), padb], axis=1).reshape(NW, CH, LW)
    r_p = jnp.concatenate(
        [receivers.reshape(NW, EWR), padb], axis=1).reshape(NW, CH, LW)

    degs_p, degr_p = _sc_degrees(s_p, r_p)

    nodes_p = jnp.pad(nodes, ((0, NP - NN), (0, 0)))
    hnorm = _tc_encoder(nodes_p, w_enc, b_enc.reshape(1, HG),
                        degs_p[0], degs_p[1])

    seg1 = _sc_segsum(hnorm, s_p, r_p, HG)

    x1 = _tc_combine1(seg1[0], seg1[1], hnorm, degr_p[0], degr_p[1])
    x = x1[:NN].reshape(B, N * HG)

    mean, log_std, zh = _tc_middle(
        x, w_fc, b_fc.reshape(1, HFC), w_mean, b_mean.reshape(1, LAT),
        w_logstd, b_logstd.reshape(1, LAT), eps, w_dec,
        b_dec.reshape(1, N * HG))

    zh3 = jnp.pad(zh.reshape(NN, HG), ((0, NP - NN), (0, 0)))
    u48 = _tc_u_table(zh3, degs_p[0], degs_p[1])

    seg2 = _sc_segsum(u48, s_p, r_p, 48)

    outp = _tc_final(seg2[0], seg2[1], w_out, b_out.reshape(1, OUT),
                     degr_p[0], degr_p[1])
    return mean, log_std, outp[:NN]


# speedup vs baseline: 9.5131x; 1.1431x over previous
import functools

import jax
import jax.numpy as jnp
from jax import lax
from jax.experimental import pallas as pl
from jax.experimental.pallas import tpu as pltpu
from jax.experimental.pallas import tpu_sc as plsc

B = 50
N = 200
HG = 32
HFC = 256
LAT = 64
OUT = 128
DFEAT = 128
E = 160000
NN = B * N
NP = 10112
RPT = 632
NW = 32
LW = 128
CH = 40
EP = NW * CH * LW
EWR = E // NW
PADW = CH * LW - EWR
GB = 16
RB = NP // GB

_SLABS = ((0, 128), (128, 128), (256, 128), (384, 128), (512, 120))

_mesh = plsc.VectorSubcoreMesh(core_axis_name="c", subcore_axis_name="s")
_sc_params = pltpu.CompilerParams(use_tc_tiling_on_sc=False)


def _sc_degrees(s_p, r_p):
    out_t = (jax.ShapeDtypeStruct((2, NP, 16), jnp.float32),
             jax.ShapeDtypeStruct((2, NP, 16), jnp.float32))

    @functools.partial(
        pl.kernel, mesh=_mesh, out_type=out_t, compiler_params=_sc_params,
        scratch_types=[
            pltpu.VMEM((CH, LW), jnp.int32),
            pltpu.VMEM((CH, LW), jnp.int32),
            pltpu.VMEM((LW, 16), jnp.float32),
            pltpu.VMEM((RPT, 16), jnp.float32),
            pltpu.VMEM_SHARED((NP, 16), jnp.float32),
            pltpu.VMEM_SHARED((NP, 16), jnp.float32),
            pltpu.SemaphoreType.DMA,
            pltpu.SemaphoreType.DMA,
        ])
    def deg_kernel(s_hbm, r_hbm, ds_out, dr_out, sidx_v, ridx_v, ones_v,
                   zero_v, ds_sh, dr_sh, dsem, rsem):
        c = lax.axis_index("c")
        s = lax.axis_index("s")
        wid = s * 2 + c
        pltpu.sync_copy(s_hbm.at[wid], sidx_v)
        pltpu.sync_copy(r_hbm.at[wid], ridx_v)

        @pl.loop(0, LW)
        def _(i):
            ones_v[i, :] = jnp.ones((16,), jnp.float32)

        @pl.loop(0, RPT)
        def _(i):
            zero_v[i, :] = jnp.zeros((16,), jnp.float32)

        base = s * RPT
        pltpu.sync_copy(zero_v, ds_sh.at[pl.ds(base, RPT)])
        pltpu.sync_copy(zero_v, dr_sh.at[pl.ds(base, RPT)])
        plsc.subcore_barrier()

        @pl.loop(0, CH)
        def _(i):
            a = pltpu.async_copy(ones_v, ds_sh.at[sidx_v.at[i]], dsem,
                                 add=True)
            b = pltpu.async_copy(ones_v, dr_sh.at[ridx_v.at[i]], rsem,
                                 add=True)
            a.wait()
            b.wait()

        plsc.subcore_barrier()
        pltpu.sync_copy(ds_sh.at[pl.ds(base, RPT)],
                        ds_out.at[c, pl.ds(base, RPT)])
        pltpu.sync_copy(dr_sh.at[pl.ds(base, RPT)],
                        dr_out.at[c, pl.ds(base, RPT)])

    return deg_kernel(s_p, r_p)


def _sc_segsum(table, s_p, r_p, d):
    @functools.partial(
        pl.kernel, mesh=_mesh,
        out_type=jax.ShapeDtypeStruct((2, NP, d), jnp.float32),
        compiler_params=_sc_params,
        scratch_types=[
            pltpu.VMEM((CH, LW), jnp.int32),
            pltpu.VMEM((CH, LW), jnp.int32),
            pltpu.VMEM((LW, d), jnp.float32),
            pltpu.VMEM((LW, d), jnp.float32),
            pltpu.VMEM_SHARED((NP, d), jnp.float32),
            pltpu.SemaphoreType.DMA,
            pltpu.SemaphoreType.DMA,
        ])
    def seg_kernel(tab_hbm, s_hbm, r_hbm, out_hbm, sidx_v, ridx_v, rows0_v,
                   rows1_v, acc_sh, sem0, sem1):
        c = lax.axis_index("c")
        s = lax.axis_index("s")
        wid = s * 2 + c
        pltpu.sync_copy(s_hbm.at[wid], sidx_v)
        pltpu.sync_copy(r_hbm.at[wid], ridx_v)

        @pl.loop(0, LW)
        def _(i):
            @pl.loop(0, d, step=16)
            def _(l):
                rows0_v[i, pl.ds(l, 16)] = jnp.zeros((16,), jnp.float32)

        base = s * RPT
        for off, sz in _SLABS:
            pltpu.sync_copy(rows0_v.at[pl.ds(0, sz)],
                            acc_sh.at[pl.ds(base + off, sz)])
        plsc.subcore_barrier()

        pltpu.async_copy(tab_hbm.at[sidx_v.at[0]], rows0_v, sem0)

        @pl.loop(0, CH // 2)
        def _(j):
            i = j * 2
            pltpu.make_async_copy(tab_hbm.at[sidx_v.at[i]], rows0_v,
                                  sem0).wait()
            pltpu.async_copy(tab_hbm.at[sidx_v.at[i + 1]], rows1_v, sem1)
            pltpu.sync_copy(rows0_v, acc_sh.at[ridx_v.at[i]], add=True)
            pltpu.make_async_copy(tab_hbm.at[sidx_v.at[i + 1]], rows1_v,
                                  sem1).wait()

            @pl.when(j < CH // 2 - 1)
            def _():
                pltpu.async_copy(tab_hbm.at[sidx_v.at[i + 2]], rows0_v, sem0)

            pltpu.sync_copy(rows1_v, acc_sh.at[ridx_v.at[i + 1]], add=True)

        plsc.subcore_barrier()
        for off, sz in _SLABS:
            pltpu.sync_copy(acc_sh.at[pl.ds(base + off, sz)],
                            out_hbm.at[c, pl.ds(base + off, sz)])

    return seg_kernel(table, s_p, r_p)


def _row_spec(width, rows=None):
    return pl.BlockSpec(((rows or RB), width), lambda i: (i, 0))


def _pair_spec(width, rows=None):
    return pl.BlockSpec((2, (rows or RB), width), lambda i: (0, i, 0))


def _full_spec(shape):
    return pl.BlockSpec(shape, lambda i: tuple(0 for _ in shape))


def _tc_encoder(nodes_p, w_enc, b_enc, ds):
    def body(n_ref, w_ref, b_ref, ds_ref, hn_ref):
        h = jnp.dot(n_ref[...], w_ref[...],
                    preferred_element_type=jnp.float32,
                    precision=lax.Precision.HIGHEST)
        h = jnp.maximum(h + b_ref[...], 0.0)
        ds = ds_ref[0] + ds_ref[1]
        is1 = lax.rsqrt(ds + 1.0)
        hn_ref[...] = h * jnp.concatenate([is1, is1], axis=1)

    return pl.pallas_call(
        body,
        grid=(GB,),
        in_specs=[_row_spec(DFEAT), _full_spec((DFEAT, HG)),
                  _full_spec((1, HG)), _pair_spec(16)],
        out_specs=_row_spec(HG),
        out_shape=jax.ShapeDtypeStruct((NP, HG), jnp.float32),
    )(nodes_p, w_enc, b_enc, ds)


def _tc_combine1(p, hn, dr):
    def body(p_ref, hn_ref, dr_ref, o_ref):
        d = dr_ref[0] + dr_ref[1]
        ir1 = lax.rsqrt(d + 1.0)
        o_ref[...] = (p_ref[0] + p_ref[1] + hn_ref[...]) * \
            jnp.concatenate([ir1, ir1], axis=1)

    return pl.pallas_call(
        body,
        grid=(10,),
        in_specs=[_pair_spec(HG, 1000), _row_spec(HG, 1000),
                  _pair_spec(16, 1000)],
        out_specs=_row_spec(HG, 1000),
        out_shape=jax.ShapeDtypeStruct((NN, HG), jnp.float32),
    )(p, hn, dr)


def _tc_middle(x, w_fc, b_fc, w_mean, b_mean, w_logstd, b_logstd, eps,
               w_dec, b_dec):
    def body(x_ref, wfc_ref, bfc_ref, wm_ref, bm_ref, wl_ref, bl_ref,
             eps_ref, wd_ref, bd_ref, mean_ref, logstd_ref, zh_ref):
        x = jnp.dot(x_ref[...], wfc_ref[...],
                    preferred_element_type=jnp.float32,
                    precision=lax.Precision.HIGHEST)
        x = jnp.maximum(x + bfc_ref[...], 0.0)
        mean = jnp.dot(x, wm_ref[...], preferred_element_type=jnp.float32,
                       precision=lax.Precision.HIGHEST) + bm_ref[...]
        logstd = jnp.dot(x, wl_ref[...], preferred_element_type=jnp.float32,
                         precision=lax.Precision.HIGHEST) + bl_ref[...]
        z = mean + jnp.exp(logstd) * eps_ref[...]
        zh = jnp.dot(z, wd_ref[...], preferred_element_type=jnp.float32,
                     precision=lax.Precision.HIGHEST)
        zh_ref[...] = jnp.maximum(zh + bd_ref[...], 0.0)
        mean_ref[...] = mean
        logstd_ref[...] = logstd

    return pl.pallas_call(
        body,
        out_shape=(jax.ShapeDtypeStruct((B, LAT), jnp.float32),
                   jax.ShapeDtypeStruct((B, LAT), jnp.float32),
                   jax.ShapeDtypeStruct((B, N * HG), jnp.float32)),
    )(x, w_fc, b_fc, w_mean, b_mean, w_logstd, b_logstd, eps, w_dec, b_dec)


def _tc_u_table(zh3, ds):
    def body(z_ref, ds_ref, o_ref):
        d = ds_ref[0] + ds_ref[1]
        iv = lax.rsqrt(jnp.maximum(d, 1.0))
        iv32 = jnp.concatenate([iv, iv], axis=1)
        o_ref[...] = jnp.concatenate([z_ref[...] * iv32, iv], axis=1)

    return pl.pallas_call(
        body,
        grid=(GB,),
        in_specs=[_row_spec(HG), _pair_spec(16)],
        out_specs=_row_spec(48),
        out_shape=jax.ShapeDtypeStruct((NP, 48), jnp.float32),
    )(zh3, ds)


def _tc_final(q, w_out, b_out, dr):
    def body(q_ref, w_ref, b_ref, dr_ref, o_ref):
        t = q_ref[0] + q_ref[1]
        u = lax.slice(t, (0, 0), (1000, HG))
        v = lax.slice(t, (0, HG), (1000, HG + 1))
        z2 = jnp.dot(u, w_ref[...], preferred_element_type=jnp.float32,
                     precision=lax.Precision.HIGHEST) + v * b_ref[...]
        d = dr_ref[0] + dr_ref[1]
        inv = lax.rsqrt(jnp.maximum(d[:, :1], 1.0))
        o_ref[...] = z2 * jnp.broadcast_to(inv, (1000, OUT))

    return pl.pallas_call(
        body,
        grid=(10,),
        in_specs=[_pair_spec(48, 1000), _full_spec((HG, OUT)),
                  _full_spec((1, OUT)), _pair_spec(16, 1000)],
        out_specs=_row_spec(OUT, 1000),
        out_shape=jax.ShapeDtypeStruct((NN, OUT), jnp.float32),
    )(q, w_out, b_out, dr)


def kernel(nodes, senders, receivers, eps, w_enc, b_enc, w_fc, b_fc, w_mean,
           b_mean, w_logstd, b_logstd, w_dec, b_dec, w_out, b_out):
    padv = NN + (jnp.arange(PADW, dtype=jnp.int32) % (NP - NN))
    padb = jnp.broadcast_to(padv[None, :], (NW, PADW))
    s_p = jnp.concatenate(
        [senders.reshape(NW, EWR), padb], axis=1).reshape(NW, CH, LW)
    r_p = jnp.concatenate(
        [receivers.reshape(NW, EWR), padb], axis=1).reshape(NW, CH, LW)

    degs_p, degr_p = _sc_degrees(s_p, r_p)

    nodes_p = jnp.pad(nodes, ((0, NP - NN), (0, 0)))
    hnorm = _tc_encoder(nodes_p, w_enc, b_enc.reshape(1, HG), degs_p)

    seg1 = _sc_segsum(hnorm, s_p, r_p, HG)

    x1 = _tc_combine1(seg1, hnorm, degr_p)
    x = x1.reshape(B, N * HG)

    mean, log_std, zh = _tc_middle(
        x, w_fc, b_fc.reshape(1, HFC), w_mean, b_mean.reshape(1, LAT),
        w_logstd, b_logstd.reshape(1, LAT), eps, w_dec,
        b_dec.reshape(1, N * HG))

    zh3 = jnp.pad(zh.reshape(NN, HG), ((0, NP - NN), (0, 0)))
    u48 = _tc_u_table(zh3, degs_p)

    seg2 = _sc_segsum(u48, s_p, r_p, 48)

    outp = _tc_final(seg2, w_out, b_out.reshape(1, OUT), degr_p)
    return mean, log_std, outp


# speedup vs baseline: 10.7682x; 1.1319x over previous
import functools

import jax
import jax.numpy as jnp
from jax import lax
from jax.experimental import pallas as pl
from jax.experimental.pallas import tpu as pltpu
from jax.experimental.pallas import tpu_sc as plsc

B = 50
N = 200
HG = 32
HFC = 256
LAT = 64
OUT = 128
DFEAT = 128
E = 160000
NN = B * N
NP = 10112
RPT = 632
NW = 32
LW = 128
CH = 40
EP = NW * CH * LW
EWR = E // NW
PADW = CH * LW - EWR
GB = 16
RB = NP // GB

_SLABS = ((0, 128), (128, 128), (256, 128), (384, 128), (512, 120))

_mesh = plsc.VectorSubcoreMesh(core_axis_name="c", subcore_axis_name="s")
_sc_params = pltpu.CompilerParams(use_tc_tiling_on_sc=False)


def _sc_degrees(s_p, r_p):
    out_t = (jax.ShapeDtypeStruct((2, NP, 16), jnp.float32),
             jax.ShapeDtypeStruct((2, NP, 16), jnp.float32))

    @functools.partial(
        pl.kernel, mesh=_mesh, out_type=out_t, compiler_params=_sc_params,
        scratch_types=[
            pltpu.VMEM((CH, LW), jnp.int32),
            pltpu.VMEM((CH, LW), jnp.int32),
            pltpu.VMEM((LW, 16), jnp.float32),
            pltpu.VMEM((RPT, 16), jnp.float32),
            pltpu.VMEM_SHARED((NP, 16), jnp.float32),
            pltpu.VMEM_SHARED((NP, 16), jnp.float32),
            pltpu.SemaphoreType.DMA,
            pltpu.SemaphoreType.DMA,
        ])
    def deg_kernel(s_hbm, r_hbm, ds_out, dr_out, sidx_v, ridx_v, ones_v,
                   zero_v, ds_sh, dr_sh, dsem, rsem):
        c = lax.axis_index("c")
        s = lax.axis_index("s")
        wid = s * 2 + c
        pltpu.sync_copy(s_hbm.at[wid], sidx_v)
        pltpu.sync_copy(r_hbm.at[wid], ridx_v)

        @pl.loop(0, LW)
        def _(i):
            ones_v[i, :] = jnp.ones((16,), jnp.float32)

        @pl.loop(0, RPT)
        def _(i):
            zero_v[i, :] = jnp.zeros((16,), jnp.float32)

        base = s * RPT
        pltpu.sync_copy(zero_v, ds_sh.at[pl.ds(base, RPT)])
        pltpu.sync_copy(zero_v, dr_sh.at[pl.ds(base, RPT)])
        plsc.subcore_barrier()

        @pl.loop(0, CH)
        def _(i):
            a = pltpu.async_copy(ones_v, ds_sh.at[sidx_v.at[i]], dsem,
                                 add=True)
            b = pltpu.async_copy(ones_v, dr_sh.at[ridx_v.at[i]], rsem,
                                 add=True)
            a.wait()
            b.wait()

        plsc.subcore_barrier()
        pltpu.sync_copy(ds_sh.at[pl.ds(base, RPT)],
                        ds_out.at[c, pl.ds(base, RPT)])
        pltpu.sync_copy(dr_sh.at[pl.ds(base, RPT)],
                        dr_out.at[c, pl.ds(base, RPT)])

    return deg_kernel(s_p, r_p)


def _sc_segsum(table, s_p, r_p, d):
    @functools.partial(
        pl.kernel, mesh=_mesh,
        out_type=jax.ShapeDtypeStruct((2, NP, d), jnp.float32),
        compiler_params=_sc_params,
        scratch_types=[
            pltpu.VMEM((CH, LW), jnp.int32),
            pltpu.VMEM((CH, LW), jnp.int32),
            pltpu.VMEM((LW, d), jnp.float32),
            pltpu.VMEM((LW, d), jnp.float32),
            pltpu.VMEM_SHARED((NP, d), jnp.float32),
            pltpu.VMEM_SHARED((NP, d), jnp.float32),
            pltpu.SemaphoreType.DMA,
            pltpu.SemaphoreType.DMA,
            pltpu.SemaphoreType.DMA,
        ])
    def seg_kernel(tab_hbm, s_hbm, r_hbm, out_hbm, sidx_v, ridx_v, rows0_v,
                   rows1_v, acc_sh, tab_sh, sem0, sem1, ssem):
        c = lax.axis_index("c")
        s = lax.axis_index("s")
        wid = s * 2 + c
        pltpu.sync_copy(s_hbm.at[wid], sidx_v)
        pltpu.sync_copy(r_hbm.at[wid], ridx_v)

        base = s * RPT
        for off, sz in _SLABS:
            pltpu.async_copy(tab_hbm.at[pl.ds(base + off, sz)],
                             tab_sh.at[pl.ds(base + off, sz)], ssem)

        @pl.loop(0, LW)
        def _(i):
            @pl.loop(0, d, step=16)
            def _(l):
                rows0_v[i, pl.ds(l, 16)] = jnp.zeros((16,), jnp.float32)

        for off, sz in _SLABS:
            pltpu.sync_copy(rows0_v.at[pl.ds(0, sz)],
                            acc_sh.at[pl.ds(base + off, sz)])
        for off, sz in _SLABS:
            pltpu.make_async_copy(tab_hbm.at[pl.ds(base + off, sz)],
                                  tab_sh.at[pl.ds(base + off, sz)],
                                  ssem).wait()
        plsc.subcore_barrier()

        pltpu.async_copy(tab_sh.at[sidx_v.at[0]], rows0_v, sem0)

        @pl.loop(0, CH // 2)
        def _(j):
            i = j * 2
            pltpu.make_async_copy(tab_sh.at[sidx_v.at[i]], rows0_v,
                                  sem0).wait()
            pltpu.async_copy(tab_sh.at[sidx_v.at[i + 1]], rows1_v, sem1)
            pltpu.sync_copy(rows0_v, acc_sh.at[ridx_v.at[i]], add=True)
            pltpu.make_async_copy(tab_sh.at[sidx_v.at[i + 1]], rows1_v,
                                  sem1).wait()

            @pl.when(j < CH // 2 - 1)
            def _():
                pltpu.async_copy(tab_sh.at[sidx_v.at[i + 2]], rows0_v, sem0)

            pltpu.sync_copy(rows1_v, acc_sh.at[ridx_v.at[i + 1]], add=True)

        plsc.subcore_barrier()
        for off, sz in _SLABS:
            pltpu.sync_copy(acc_sh.at[pl.ds(base + off, sz)],
                            out_hbm.at[c, pl.ds(base + off, sz)])

    return seg_kernel(table, s_p, r_p)


def _row_spec(width, rows=None):
    return pl.BlockSpec(((rows or RB), width), lambda i: (i, 0))


def _pair_spec(width, rows=None):
    return pl.BlockSpec((2, (rows or RB), width), lambda i: (0, i, 0))


def _full_spec(shape):
    return pl.BlockSpec(shape, lambda i: tuple(0 for _ in shape))


def _tc_encoder(nodes_p, w_enc, b_enc, ds):
    def body(n_ref, w_ref, b_ref, ds_ref, hn_ref):
        h = jnp.dot(n_ref[...], w_ref[...],
                    preferred_element_type=jnp.float32,
                    precision=lax.Precision.HIGHEST)
        h = jnp.maximum(h + b_ref[...], 0.0)
        ds = ds_ref[0] + ds_ref[1]
        is1 = lax.rsqrt(ds + 1.0)
        hn_ref[...] = h * jnp.concatenate([is1, is1], axis=1)

    return pl.pallas_call(
        body,
        grid=(GB,),
        in_specs=[_row_spec(DFEAT), _full_spec((DFEAT, HG)),
                  _full_spec((1, HG)), _pair_spec(16)],
        out_specs=_row_spec(HG),
        out_shape=jax.ShapeDtypeStruct((NP, HG), jnp.float32),
    )(nodes_p, w_enc, b_enc, ds)


def _tc_combine1(p, hn, dr):
    def body(p_ref, hn_ref, dr_ref, o_ref):
        d = dr_ref[0] + dr_ref[1]
        ir1 = lax.rsqrt(d + 1.0)
        o_ref[...] = (p_ref[0] + p_ref[1] + hn_ref[...]) * \
            jnp.concatenate([ir1, ir1], axis=1)

    return pl.pallas_call(
        body,
        grid=(10,),
        in_specs=[_pair_spec(HG, 1000), _row_spec(HG, 1000),
                  _pair_spec(16, 1000)],
        out_specs=_row_spec(HG, 1000),
        out_shape=jax.ShapeDtypeStruct((NN, HG), jnp.float32),
    )(p, hn, dr)


def _tc_middle(x, w_fc, b_fc, w_mean, b_mean, w_logstd, b_logstd, eps,
               w_dec, b_dec):
    def body(x_ref, wfc_ref, bfc_ref, wm_ref, bm_ref, wl_ref, bl_ref,
             eps_ref, wd_ref, bd_ref, mean_ref, logstd_ref, zh_ref):
        x = jnp.dot(x_ref[...], wfc_ref[...],
                    preferred_element_type=jnp.float32,
                    precision=lax.Precision.HIGHEST)
        x = jnp.maximum(x + bfc_ref[...], 0.0)
        mean = jnp.dot(x, wm_ref[...], preferred_element_type=jnp.float32,
                       precision=lax.Precision.HIGHEST) + bm_ref[...]
        logstd = jnp.dot(x, wl_ref[...], preferred_element_type=jnp.float32,
                         precision=lax.Precision.HIGHEST) + bl_ref[...]
        z = mean + jnp.exp(logstd) * eps_ref[...]
        zh = jnp.dot(z, wd_ref[...], preferred_element_type=jnp.float32,
                     precision=lax.Precision.HIGHEST)
        zh_ref[...] = jnp.maximum(zh + bd_ref[...], 0.0)
        mean_ref[...] = mean
        logstd_ref[...] = logstd

    return pl.pallas_call(
        body,
        out_shape=(jax.ShapeDtypeStruct((B, LAT), jnp.float32),
                   jax.ShapeDtypeStruct((B, LAT), jnp.float32),
                   jax.ShapeDtypeStruct((B, N * HG), jnp.float32)),
    )(x, w_fc, b_fc, w_mean, b_mean, w_logstd, b_logstd, eps, w_dec, b_dec)


def _tc_u_table(zh3, ds):
    def body(z_ref, ds_ref, o_ref):
        d = ds_ref[0] + ds_ref[1]
        iv = lax.rsqrt(jnp.maximum(d, 1.0))
        iv32 = jnp.concatenate([iv, iv], axis=1)
        o_ref[...] = jnp.concatenate([z_ref[...] * iv32, iv], axis=1)

    return pl.pallas_call(
        body,
        grid=(GB,),
        in_specs=[_row_spec(HG), _pair_spec(16)],
        out_specs=_row_spec(48),
        out_shape=jax.ShapeDtypeStruct((NP, 48), jnp.float32),
    )(zh3, ds)


def _tc_final(q, w_out, b_out, dr):
    def body(q_ref, w_ref, b_ref, dr_ref, o_ref):
        t = q_ref[0] + q_ref[1]
        u = lax.slice(t, (0, 0), (1000, HG))
        v = lax.slice(t, (0, HG), (1000, HG + 1))
        z2 = jnp.dot(u, w_ref[...], preferred_element_type=jnp.float32,
                     precision=lax.Precision.HIGHEST) + v * b_ref[...]
        d = dr_ref[0] + dr_ref[1]
        inv = lax.rsqrt(jnp.maximum(d[:, :1], 1.0))
        o_ref[...] = z2 * jnp.broadcast_to(inv, (1000, OUT))

    return pl.pallas_call(
        body,
        grid=(10,),
        in_specs=[_pair_spec(48, 1000), _full_spec((HG, OUT)),
                  _full_spec((1, OUT)), _pair_spec(16, 1000)],
        out_specs=_row_spec(OUT, 1000),
        out_shape=jax.ShapeDtypeStruct((NN, OUT), jnp.float32),
    )(q, w_out, b_out, dr)


def kernel(nodes, senders, receivers, eps, w_enc, b_enc, w_fc, b_fc, w_mean,
           b_mean, w_logstd, b_logstd, w_dec, b_dec, w_out, b_out):
    padv = NN + (jnp.arange(PADW, dtype=jnp.int32) % (NP - NN))
    padb = jnp.broadcast_to(padv[None, :], (NW, PADW))
    s_p = jnp.concatenate(
        [senders.reshape(NW, EWR), padb], axis=1).reshape(NW, CH, LW)
    r_p = jnp.concatenate(
        [receivers.reshape(NW, EWR), padb], axis=1).reshape(NW, CH, LW)

    degs_p, degr_p = _sc_degrees(s_p, r_p)

    nodes_p = jnp.pad(nodes, ((0, NP - NN), (0, 0)))
    hnorm = _tc_encoder(nodes_p, w_enc, b_enc.reshape(1, HG), degs_p)

    seg1 = _sc_segsum(hnorm, s_p, r_p, HG)

    x1 = _tc_combine1(seg1, hnorm, degr_p)
    x = x1.reshape(B, N * HG)

    mean, log_std, zh = _tc_middle(
        x, w_fc, b_fc.reshape(1, HFC), w_mean, b_mean.reshape(1, LAT),
        w_logstd, b_logstd.reshape(1, LAT), eps, w_dec,
        b_dec.reshape(1, N * HG))

    zh3 = jnp.pad(zh.reshape(NN, HG), ((0, NP - NN), (0, 0)))
    u48 = _tc_u_table(zh3, degs_p)

    seg2 = _sc_segsum(u48, s_p, r_p, 48)

    outp = _tc_final(seg2, w_out, b_out.reshape(1, OUT), degr_p)
    return mean, log_std, outp


# speedup vs baseline: 11.3763x; 1.0565x over previous
import functools

import jax
import jax.numpy as jnp
from jax import lax
from jax.experimental import pallas as pl
from jax.experimental.pallas import tpu as pltpu
from jax.experimental.pallas import tpu_sc as plsc

B = 50
N = 200
HG = 32
HFC = 256
LAT = 64
OUT = 128
DFEAT = 128
E = 160000
NN = B * N
NP = 10112
RPT = 632
NW = 32
LW = 128
CH = 40
EP = NW * CH * LW
EWR = E // NW
PADW = CH * LW - EWR
GB = 16
RB = NP // GB

_SLABS = ((0, 128), (128, 128), (256, 128), (384, 128), (512, 120))

_mesh = plsc.VectorSubcoreMesh(core_axis_name="c", subcore_axis_name="s")
_sc_params = pltpu.CompilerParams(use_tc_tiling_on_sc=False)


def _sc_degrees(s_p, r_p):
    out_t = (jax.ShapeDtypeStruct((2, NP, 16), jnp.float32),
             jax.ShapeDtypeStruct((2, NP, 16), jnp.float32))

    @functools.partial(
        pl.kernel, mesh=_mesh, out_type=out_t, compiler_params=_sc_params,
        scratch_types=[
            pltpu.VMEM((CH, LW), jnp.int32),
            pltpu.VMEM((CH, LW), jnp.int32),
            pltpu.VMEM((LW, 16), jnp.float32),
            pltpu.VMEM((RPT, 16), jnp.float32),
            pltpu.VMEM_SHARED((NP, 16), jnp.float32),
            pltpu.VMEM_SHARED((NP, 16), jnp.float32),
            pltpu.SemaphoreType.DMA,
            pltpu.SemaphoreType.DMA,
        ])
    def deg_kernel(s_hbm, r_hbm, ds_out, dr_out, sidx_v, ridx_v, ones_v,
                   zero_v, ds_sh, dr_sh, dsem, rsem):
        c = lax.axis_index("c")
        s = lax.axis_index("s")
        wid = s * 2 + c
        pltpu.sync_copy(s_hbm.at[wid], sidx_v)
        pltpu.sync_copy(r_hbm.at[wid], ridx_v)

        @pl.loop(0, LW)
        def _(i):
            ones_v[i, :] = jnp.ones((16,), jnp.float32)

        @pl.loop(0, RPT)
        def _(i):
            zero_v[i, :] = jnp.zeros((16,), jnp.float32)

        base = s * RPT
        pltpu.sync_copy(zero_v, ds_sh.at[pl.ds(base, RPT)])
        pltpu.sync_copy(zero_v, dr_sh.at[pl.ds(base, RPT)])
        plsc.subcore_barrier()

        @pl.loop(0, CH // 8)
        def _(bb):
            i0 = bb * 8
            for k in range(8):
                pltpu.async_copy(ones_v, ds_sh.at[sidx_v.at[i0 + k]], dsem,
                                 add=True)
                pltpu.async_copy(ones_v, dr_sh.at[ridx_v.at[i0 + k]], rsem,
                                 add=True)
            for k in range(8):
                pltpu.make_async_copy(ones_v, ds_sh.at[sidx_v.at[i0 + k]],
                                      dsem).wait()
                pltpu.make_async_copy(ones_v, dr_sh.at[ridx_v.at[i0 + k]],
                                      rsem).wait()

        plsc.subcore_barrier()
        pltpu.sync_copy(ds_sh.at[pl.ds(base, RPT)],
                        ds_out.at[c, pl.ds(base, RPT)])
        pltpu.sync_copy(dr_sh.at[pl.ds(base, RPT)],
                        dr_out.at[c, pl.ds(base, RPT)])

    return deg_kernel(s_p, r_p)


def _sc_segsum(table, s_p, r_p, d):
    @functools.partial(
        pl.kernel, mesh=_mesh,
        out_type=jax.ShapeDtypeStruct((2, NP, d), jnp.float32),
        compiler_params=_sc_params,
        scratch_types=[
            pltpu.VMEM((CH, LW), jnp.int32),
            pltpu.VMEM((CH, LW), jnp.int32),
            pltpu.VMEM((4, LW, d), jnp.float32),
            pltpu.VMEM_SHARED((NP, d), jnp.float32),
            pltpu.VMEM_SHARED((NP, d), jnp.float32),
            pltpu.SemaphoreType.DMA,
            pltpu.SemaphoreType.DMA,
            pltpu.SemaphoreType.DMA,
            pltpu.SemaphoreType.DMA,
            pltpu.SemaphoreType.DMA,
            pltpu.SemaphoreType.DMA,
            pltpu.SemaphoreType.DMA,
            pltpu.SemaphoreType.DMA,
            pltpu.SemaphoreType.DMA,
        ])
    def seg_kernel(tab_hbm, s_hbm, r_hbm, out_hbm, sidx_v, ridx_v, rows_v,
                   acc_sh, tab_sh, gs0, gs1, gs2, gs3, ss0, ss1, ss2, ss3,
                   ssem):
        c = lax.axis_index("c")
        s = lax.axis_index("s")
        wid = s * 2 + c
        pltpu.sync_copy(s_hbm.at[wid], sidx_v)
        pltpu.sync_copy(r_hbm.at[wid], ridx_v)

        base = s * RPT
        for off, sz in _SLABS:
            pltpu.async_copy(tab_hbm.at[pl.ds(base + off, sz)],
                             tab_sh.at[pl.ds(base + off, sz)], ssem)

        @pl.loop(0, LW)
        def _(i):
            @pl.loop(0, d, step=16)
            def _(l):
                rows_v[0, i, pl.ds(l, 16)] = jnp.zeros((16,), jnp.float32)

        for off, sz in _SLABS:
            pltpu.sync_copy(rows_v.at[0, pl.ds(0, sz)],
                            acc_sh.at[pl.ds(base + off, sz)])
        for off, sz in _SLABS:
            pltpu.make_async_copy(tab_hbm.at[pl.ds(base + off, sz)],
                                  tab_sh.at[pl.ds(base + off, sz)],
                                  ssem).wait()
        plsc.subcore_barrier()

        gsems = (gs0, gs1, gs2, gs3)
        ssems = (ss0, ss1, ss2, ss3)
        for b in range(4):
            pltpu.async_copy(tab_sh.at[sidx_v.at[b]], rows_v.at[b], gsems[b])

        @pl.loop(0, CH // 4)
        def _(jj):
            i0 = jj * 4
            for b in range(4):
                i = i0 + b
                pltpu.make_async_copy(tab_sh.at[sidx_v.at[i]], rows_v.at[b],
                                      gsems[b]).wait()
                pltpu.async_copy(rows_v.at[b], acc_sh.at[ridx_v.at[i]],
                                 ssems[b], add=True)
                pb = (b + 3) % 4

                @pl.when((i >= 1) & (i < CH - 3))
                def _():
                    pltpu.make_async_copy(rows_v.at[pb],
                                          acc_sh.at[ridx_v.at[i - 1]],
                                          ssems[pb]).wait()
                    pltpu.async_copy(tab_sh.at[sidx_v.at[i + 3]],
                                     rows_v.at[pb], gsems[pb])

        for b in range(4):
            i = CH - 4 + b
            pltpu.make_async_copy(rows_v.at[b], acc_sh.at[ridx_v.at[i]],
                                  ssems[b]).wait()
        plsc.subcore_barrier()
        for off, sz in _SLABS:
            pltpu.sync_copy(acc_sh.at[pl.ds(base + off, sz)],
                            out_hbm.at[c, pl.ds(base + off, sz)])

    return seg_kernel(table, s_p, r_p)


def _row_spec(width, rows=None):
    return pl.BlockSpec(((rows or RB), width), lambda i: (i, 0))


def _pair_spec(width, rows=None):
    return pl.BlockSpec((2, (rows or RB), width), lambda i: (0, i, 0))


def _full_spec(shape):
    return pl.BlockSpec(shape, lambda i: tuple(0 for _ in shape))


def _tc_h(nodes_p, w_enc, b_enc):
    def body(n_ref, w_ref, b_ref, h_ref):
        h = jnp.dot(n_ref[...], w_ref[...],
                    preferred_element_type=jnp.float32,
                    precision=lax.Precision.HIGHEST)
        h_ref[...] = jnp.maximum(h + b_ref[...], 0.0)

    return pl.pallas_call(
        body,
        grid=(GB,),
        in_specs=[_row_spec(DFEAT), _full_spec((DFEAT, HG)),
                  _full_spec((1, HG))],
        out_specs=_row_spec(HG),
        out_shape=jax.ShapeDtypeStruct((NP, HG), jnp.float32),
    )(nodes_p, w_enc, b_enc)


def _tc_scale_h(h, ds):
    def body(h_ref, ds_ref, hn_ref):
        d = ds_ref[0] + ds_ref[1]
        is1 = lax.rsqrt(d + 1.0)
        hn_ref[...] = h_ref[...] * jnp.concatenate([is1, is1], axis=1)

    return pl.pallas_call(
        body,
        grid=(GB,),
        in_specs=[_row_spec(HG), _pair_spec(16)],
        out_specs=_row_spec(HG),
        out_shape=jax.ShapeDtypeStruct((NP, HG), jnp.float32),
    )(h, ds)


def _tc_combine1(p, hn, dr):
    def body(p_ref, hn_ref, dr_ref, o_ref):
        d = dr_ref[0] + dr_ref[1]
        ir1 = lax.rsqrt(d + 1.0)
        o_ref[...] = (p_ref[0] + p_ref[1] + hn_ref[...]) * \
            jnp.concatenate([ir1, ir1], axis=1)

    return pl.pallas_call(
        body,
        grid=(10,),
        in_specs=[_pair_spec(HG, 1000), _row_spec(HG, 1000),
                  _pair_spec(16, 1000)],
        out_specs=_row_spec(HG, 1000),
        out_shape=jax.ShapeDtypeStruct((NN, HG), jnp.float32),
    )(p, hn, dr)


def _tc_middle(x, w_fc, b_fc, w_mean, b_mean, w_logstd, b_logstd, eps,
               w_dec, b_dec):
    def body(x_ref, wfc_ref, bfc_ref, wm_ref, bm_ref, wl_ref, bl_ref,
             eps_ref, wd_ref, bd_ref, mean_ref, logstd_ref, zh_ref):
        x = jnp.dot(x_ref[...], wfc_ref[...],
                    preferred_element_type=jnp.float32,
                    precision=lax.Precision.HIGHEST)
        x = jnp.maximum(x + bfc_ref[...], 0.0)
        mean = jnp.dot(x, wm_ref[...], preferred_element_type=jnp.float32,
                       precision=lax.Precision.HIGHEST) + bm_ref[...]
        logstd = jnp.dot(x, wl_ref[...], preferred_element_type=jnp.float32,
                         precision=lax.Precision.HIGHEST) + bl_ref[...]
        z = mean + jnp.exp(logstd) * eps_ref[...]
        zh = jnp.dot(z, wd_ref[...], preferred_element_type=jnp.float32,
                     precision=lax.Precision.HIGHEST)
        zh_ref[...] = jnp.maximum(zh + bd_ref[...], 0.0)
        mean_ref[...] = mean
        logstd_ref[...] = logstd

    return pl.pallas_call(
        body,
        out_shape=(jax.ShapeDtypeStruct((B, LAT), jnp.float32),
                   jax.ShapeDtypeStruct((B, LAT), jnp.float32),
                   jax.ShapeDtypeStruct((B, N * HG), jnp.float32)),
    )(x, w_fc, b_fc, w_mean, b_mean, w_logstd, b_logstd, eps, w_dec, b_dec)


def _tc_u_table(zh3, ds):
    def body(z_ref, ds_ref, o_ref):
        d = ds_ref[0] + ds_ref[1]
        iv = lax.rsqrt(jnp.maximum(d, 1.0))
        iv32 = jnp.concatenate([iv, iv], axis=1)
        o_ref[...] = jnp.concatenate([z_ref[...] * iv32, iv], axis=1)

    return pl.pallas_call(
        body,
        grid=(GB,),
        in_specs=[_row_spec(HG), _pair_spec(16)],
        out_specs=_row_spec(48),
        out_shape=jax.ShapeDtypeStruct((NP, 48), jnp.float32),
    )(zh3, ds)


def _tc_final(q, w_out, b_out, dr):
    def body(q_ref, w_ref, b_ref, dr_ref, o_ref):
        t = q_ref[0] + q_ref[1]
        u = lax.slice(t, (0, 0), (1000, HG))
        v = lax.slice(t, (0, HG), (1000, HG + 1))
        z2 = jnp.dot(u, w_ref[...], preferred_element_type=jnp.float32,
                     precision=lax.Precision.HIGHEST) + v * b_ref[...]
        d = dr_ref[0] + dr_ref[1]
        inv = lax.rsqrt(jnp.maximum(d[:, :1], 1.0))
        o_ref[...] = z2 * jnp.broadcast_to(inv, (1000, OUT))

    return pl.pallas_call(
        body,
        grid=(10,),
        in_specs=[_pair_spec(48, 1000), _full_spec((HG, OUT)),
                  _full_spec((1, OUT)), _pair_spec(16, 1000)],
        out_specs=_row_spec(OUT, 1000),
        out_shape=jax.ShapeDtypeStruct((NN, OUT), jnp.float32),
    )(q, w_out, b_out, dr)


def kernel(nodes, senders, receivers, eps, w_enc, b_enc, w_fc, b_fc, w_mean,
           b_mean, w_logstd, b_logstd, w_dec, b_dec, w_out, b_out):
    padv = NN + (jnp.arange(PADW, dtype=jnp.int32) % (NP - NN))
    padb = jnp.broadcast_to(padv[None, :], (NW, PADW))
    s_p = jnp.concatenate(
        [senders.reshape(NW, EWR), padb], axis=1).reshape(NW, CH, LW)
    r_p = jnp.concatenate(
        [receivers.reshape(NW, EWR), padb], axis=1).reshape(NW, CH, LW)

    degs_p, degr_p = _sc_degrees(s_p, r_p)

    nodes_p = jnp.pad(nodes, ((0, NP - NN), (0, 0)))
    h = _tc_h(nodes_p, w_enc, b_enc.reshape(1, HG))
    hnorm = _tc_scale_h(h, degs_p)

    seg1 = _sc_segsum(hnorm, s_p, r_p, HG)

    x1 = _tc_combine1(seg1, hnorm, degr_p)
    x = x1.reshape(B, N * HG)

    mean, log_std, zh = _tc_middle(
        x, w_fc, b_fc.reshape(1, HFC), w_mean, b_mean.reshape(1, LAT),
        w_logstd, b_logstd.reshape(1, LAT), eps, w_dec,
        b_dec.reshape(1, N * HG))

    zh3 = jnp.pad(zh.reshape(NN, HG), ((0, NP - NN), (0, 0)))
    u48 = _tc_u_table(zh3, degs_p)

    seg2 = _sc_segsum(u48, s_p, r_p, 48)

    outp = _tc_final(seg2, w_out, b_out.reshape(1, OUT), degr_p)
    return mean, log_std, outp


# speedup vs baseline: 12.8203x; 1.1269x over previous
import functools

import jax
import jax.numpy as jnp
from jax import lax
from jax.experimental import pallas as pl
from jax.experimental.pallas import tpu as pltpu
from jax.experimental.pallas import tpu_sc as plsc

B = 50
N = 200
HG = 32
HFC = 256
LAT = 64
OUT = 128
DFEAT = 128
E = 160000
NN = B * N
NP = 10112
RPT = 632
GPT = RPT // 8
NPK = NP // 8
NW = 32
LW = 128
CH = 40
EP = NW * CH * LW
EWR = E // NW
PADW = CH * LW - EWR
GB = 16
RB = NP // GB

_SLABS = ((0, 128), (128, 128), (256, 128), (384, 128), (512, 120))

_mesh = plsc.VectorSubcoreMesh(core_axis_name="c", subcore_axis_name="s")
_sc_params = pltpu.CompilerParams(use_tc_tiling_on_sc=False)


def _sc_degrees(s_p, r_p):
    out_t = (jax.ShapeDtypeStruct((2, NP, 16), jnp.float32),
             jax.ShapeDtypeStruct((2, NP, 16), jnp.float32))

    @functools.partial(
        pl.kernel, mesh=_mesh, out_type=out_t, compiler_params=_sc_params,
        scratch_types=[
            pltpu.VMEM((CH, LW), jnp.int32),
            pltpu.VMEM((CH, LW), jnp.int32),
            pltpu.VMEM((LW, 16), jnp.float32),
            pltpu.VMEM((RPT, 16), jnp.float32),
            pltpu.VMEM_SHARED((NP, 16), jnp.float32),
            pltpu.VMEM_SHARED((NP, 16), jnp.float32),
            pltpu.SemaphoreType.DMA,
            pltpu.SemaphoreType.DMA,
        ])
    def deg_kernel(s_hbm, r_hbm, ds_out, dr_out, sidx_v, ridx_v, ones_v,
                   zero_v, ds_sh, dr_sh, dsem, rsem):
        c = lax.axis_index("c")
        s = lax.axis_index("s")
        wid = s * 2 + c
        pltpu.sync_copy(s_hbm.at[wid], sidx_v)
        pltpu.sync_copy(r_hbm.at[wid], ridx_v)

        @pl.loop(0, LW)
        def _(i):
            ones_v[i, :] = jnp.ones((16,), jnp.float32)

        @pl.loop(0, RPT)
        def _(i):
            zero_v[i, :] = jnp.zeros((16,), jnp.float32)

        base = s * RPT
        pltpu.sync_copy(zero_v, ds_sh.at[pl.ds(base, RPT)])
        pltpu.sync_copy(zero_v, dr_sh.at[pl.ds(base, RPT)])
        plsc.subcore_barrier()

        @pl.loop(0, CH // 8)
        def _(bb):
            i0 = bb * 8
            for k in range(8):
                pltpu.async_copy(ones_v, ds_sh.at[sidx_v.at[i0 + k]], dsem,
                                 add=True)
                pltpu.async_copy(ones_v, dr_sh.at[ridx_v.at[i0 + k]], rsem,
                                 add=True)
            for k in range(8):
                pltpu.make_async_copy(ones_v, ds_sh.at[sidx_v.at[i0 + k]],
                                      dsem).wait()
                pltpu.make_async_copy(ones_v, dr_sh.at[ridx_v.at[i0 + k]],
                                      rsem).wait()

        plsc.subcore_barrier()
        pltpu.sync_copy(ds_sh.at[pl.ds(base, RPT)],
                        ds_out.at[c, pl.ds(base, RPT)])
        pltpu.sync_copy(dr_sh.at[pl.ds(base, RPT)],
                        dr_out.at[c, pl.ds(base, RPT)])

    return deg_kernel(s_p, r_p)


def _scale_rows(dst, src, sbuf, nlanes):
    @pl.loop(0, GPT)
    def _(g):
        row0 = g * 8
        for k in range(8):
            sv = sbuf[g, pl.ds(16 * k, 16)]
            for h in range(0, nlanes, 16):
                dst[row0 + k, pl.ds(h, 16)] = \
                    src[row0 + k, pl.ds(h, 16)] * sv


def _sc_gcn1(h, is1_pk, ir1_pk, s_p, r_p):
    d = HG

    @functools.partial(
        pl.kernel, mesh=_mesh,
        out_type=jax.ShapeDtypeStruct((2, NP, d), jnp.float32),
        compiler_params=_sc_params,
        scratch_types=[
            pltpu.VMEM((CH, LW), jnp.int32),
            pltpu.VMEM((CH, LW), jnp.int32),
            pltpu.VMEM((4, LW, d), jnp.float32),
            pltpu.VMEM((RPT, d), jnp.float32),
            pltpu.VMEM((GPT, 128), jnp.float32),
            pltpu.VMEM_SHARED((NP, d), jnp.float32),
            pltpu.VMEM_SHARED((NP, d), jnp.float32),
        ] + [pltpu.SemaphoreType.DMA] * 8)
    def k1(h_hbm, is1_hbm, ir1_hbm, s_hbm, r_hbm, out_hbm, sidx_v, ridx_v,
           rows_v, nbuf, sbuf, acc_sh, tab_sh, gs0, gs1, gs2, gs3, ss0, ss1,
           ss2, ss3):
        c = lax.axis_index("c")
        s = lax.axis_index("s")
        wid = s * 2 + c
        base = s * RPT
        pltpu.sync_copy(s_hbm.at[wid], sidx_v)
        pltpu.sync_copy(r_hbm.at[wid], ridx_v)
        pltpu.sync_copy(h_hbm.at[pl.ds(base, RPT)], nbuf)
        pltpu.sync_copy(is1_hbm.at[pl.ds(s * GPT, GPT)], sbuf)
        _scale_rows(nbuf, nbuf, sbuf, d)
        pltpu.sync_copy(nbuf, tab_sh.at[pl.ds(base, RPT)])

        @pl.when(c == 0)
        def _():
            pltpu.sync_copy(nbuf, acc_sh.at[pl.ds(base, RPT)])

        @pl.when(c == 1)
        def _():
            @pl.loop(0, LW)
            def _(i):
                @pl.loop(0, d, step=16)
                def _(l):
                    rows_v[0, i, pl.ds(l, 16)] = jnp.zeros((16,),
                                                           jnp.float32)

            for off, sz in _SLABS:
                pltpu.sync_copy(rows_v.at[0, pl.ds(0, sz)],
                                acc_sh.at[pl.ds(base + off, sz)])

        plsc.subcore_barrier()

        gsems = (gs0, gs1, gs2, gs3)
        ssems = (ss0, ss1, ss2, ss3)
        for b in range(4):
            pltpu.async_copy(tab_sh.at[sidx_v.at[b]], rows_v.at[b], gsems[b])

        @pl.loop(0, CH // 4)
        def _(jj):
            i0 = jj * 4
            for b in range(4):
                i = i0 + b
                pltpu.make_async_copy(tab_sh.at[sidx_v.at[i]], rows_v.at[b],
                                      gsems[b]).wait()
                pltpu.async_copy(rows_v.at[b], acc_sh.at[ridx_v.at[i]],
                                 ssems[b], add=True)
                pb = (b + 3) % 4

                @pl.when((i >= 1) & (i < CH - 3))
                def _():
                    pltpu.make_async_copy(rows_v.at[pb],
                                          acc_sh.at[ridx_v.at[i - 1]],
                                          ssems[pb]).wait()
                    pltpu.async_copy(tab_sh.at[sidx_v.at[i + 3]],
                                     rows_v.at[pb], gsems[pb])

        for b in range(4):
            i = CH - 4 + b
            pltpu.make_async_copy(rows_v.at[b], acc_sh.at[ridx_v.at[i]],
                                  ssems[b]).wait()
        plsc.subcore_barrier()

        pltpu.sync_copy(acc_sh.at[pl.ds(base, RPT)], nbuf)
        pltpu.sync_copy(ir1_hbm.at[pl.ds(s * GPT, GPT)], sbuf)
        _scale_rows(nbuf, nbuf, sbuf, d)
        pltpu.sync_copy(nbuf, out_hbm.at[c, pl.ds(base, RPT)])

    return k1(h, is1_pk, ir1_pk, s_p, r_p)


def _sc_gcn2(zh3, is2_pk, ir2_pk, s_p, r_p):
    d = 48
    HR0, HR1 = 320, RPT - 320
    HG0, HG1 = HR0 // 8, HR1 // 8

    @functools.partial(
        pl.kernel, mesh=_mesh,
        out_type=jax.ShapeDtypeStruct((2, NP, d), jnp.float32),
        compiler_params=_sc_params,
        scratch_types=[
            pltpu.VMEM((CH, LW), jnp.int32),
            pltpu.VMEM((CH, LW), jnp.int32),
            pltpu.VMEM((4, LW, d), jnp.float32),
            pltpu.VMEM((HR0, HG), jnp.float32),
            pltpu.VMEM((HR0, d), jnp.float32),
            pltpu.VMEM((HG0, 128), jnp.float32),
            pltpu.VMEM_SHARED((NP, d), jnp.float32),
            pltpu.VMEM_SHARED((NP, d), jnp.float32),
        ] + [pltpu.SemaphoreType.DMA] * 8)
    def k2(z_hbm, is2_hbm, ir2_hbm, s_hbm, r_hbm, out_hbm, sidx_v, ridx_v,
           rows_v, zbuf, ubuf, sbuf, acc_sh, tab_sh, gs0, gs1, gs2, gs3,
           ss0, ss1, ss2, ss3):
        c = lax.axis_index("c")
        s = lax.axis_index("s")
        wid = s * 2 + c
        base = s * RPT
        pltpu.sync_copy(s_hbm.at[wid], sidx_v)
        pltpu.sync_copy(r_hbm.at[wid], ridx_v)

        if True:
            for p, (hoff, hr, hgrp) in enumerate(((0, HR0, HG0),
                                                  (HR0, HR1, HG1))):
                b0 = base + hoff
                if p == 0:
                    pltpu.sync_copy(z_hbm.at[pl.ds(b0, hr)],
                                    zbuf.at[pl.ds(0, hr)])
                else:
                    @pl.when(s < 15)
                    def _():
                        pltpu.sync_copy(z_hbm.at[pl.ds(b0, hr)],
                                        zbuf.at[pl.ds(0, hr)])

                    @pl.when(s == 15)
                    def _():
                        pltpu.sync_copy(
                            z_hbm.at[pl.ds(b0, NN - 15 * RPT - HR0)],
                            zbuf.at[pl.ds(0, NN - 15 * RPT - HR0)])
                pltpu.sync_copy(
                    is2_hbm.at[pl.ds(s * GPT + hoff // 8, hgrp)],
                    sbuf.at[pl.ds(0, hgrp)])

                @pl.loop(0, hgrp)
                def _(g):
                    row0 = g * 8
                    for k in range(8):
                        sv = sbuf[g, pl.ds(16 * k, 16)]
                        for hh in range(0, HG, 16):
                            ubuf[row0 + k, pl.ds(hh, 16)] = \
                                zbuf[row0 + k, pl.ds(hh, 16)] * sv
                        ubuf[row0 + k, pl.ds(HG, 16)] = sv

                pltpu.sync_copy(ubuf.at[pl.ds(0, hr)],
                                tab_sh.at[pl.ds(b0, hr)])

            @pl.loop(0, LW)
            def _(i):
                @pl.loop(0, d, step=16)
                def _(l):
                    rows_v[0, i, pl.ds(l, 16)] = jnp.zeros((16,),
                                                           jnp.float32)

            for off, sz in _SLABS:
                pltpu.sync_copy(rows_v.at[0, pl.ds(0, sz)],
                                acc_sh.at[pl.ds(base + off, sz)])
            plsc.subcore_barrier()

            gsems = (gs0, gs1, gs2, gs3)
            ssems = (ss0, ss1, ss2, ss3)
            for b in range(4):
                pltpu.async_copy(tab_sh.at[sidx_v.at[b]], rows_v.at[b],
                                 gsems[b])

            @pl.loop(0, CH // 4)
            def _(jj):
                i0 = jj * 4
                for b in range(4):
                    i = i0 + b
                    pltpu.make_async_copy(tab_sh.at[sidx_v.at[i]],
                                          rows_v.at[b], gsems[b]).wait()
                    pltpu.async_copy(rows_v.at[b], acc_sh.at[ridx_v.at[i]],
                                     ssems[b], add=True)
                    pb = (b + 3) % 4

                    @pl.when((i >= 1) & (i < CH - 3))
                    def _():
                        pltpu.make_async_copy(rows_v.at[pb],
                                              acc_sh.at[ridx_v.at[i - 1]],
                                              ssems[pb]).wait()
                        pltpu.async_copy(tab_sh.at[sidx_v.at[i + 3]],
                                         rows_v.at[pb], gsems[pb])

            for b in range(4):
                i = CH - 4 + b
                pltpu.make_async_copy(rows_v.at[b], acc_sh.at[ridx_v.at[i]],
                                      ssems[b]).wait()
            plsc.subcore_barrier()

            for p, (hoff, hr, hgrp) in enumerate(((0, HR0, HG0),
                                                  (HR0, HR1, HG1))):
                b0 = base + hoff
                pltpu.sync_copy(acc_sh.at[pl.ds(b0, hr)],
                                ubuf.at[pl.ds(0, hr)])
                pltpu.sync_copy(
                    ir2_hbm.at[pl.ds(s * GPT + hoff // 8, hgrp)],
                    sbuf.at[pl.ds(0, hgrp)])

                @pl.loop(0, hgrp)
                def _(g):
                    row0 = g * 8
                    for k in range(8):
                        sv = sbuf[g, pl.ds(16 * k, 16)]
                        for hh in range(0, d, 16):
                            ubuf[row0 + k, pl.ds(hh, 16)] = \
                                ubuf[row0 + k, pl.ds(hh, 16)] * sv

                pltpu.sync_copy(ubuf.at[pl.ds(0, hr)],
                                out_hbm.at[c, pl.ds(b0, hr)])

    return k2(zh3, is2_pk, ir2_pk, s_p, r_p)


def _row_spec(width, rows=None):
    return pl.BlockSpec(((rows or RB), width), lambda i: (i, 0))


def _pair_spec(width, rows=None):
    return pl.BlockSpec((2, (rows or RB), width), lambda i: (0, i, 0))


def _full_spec(shape):
    return pl.BlockSpec(shape, lambda i: tuple(0 for _ in shape))


def _tc_h(nodes_p, w_enc, b_enc):
    def body(n_ref, w_ref, b_ref, h_ref):
        h = jnp.dot(n_ref[...], w_ref[...],
                    preferred_element_type=jnp.float32,
                    precision=lax.Precision.HIGHEST)
        h_ref[...] = jnp.maximum(h + b_ref[...], 0.0)

    return pl.pallas_call(
        body,
        grid=(GB,),
        in_specs=[_row_spec(DFEAT), _full_spec((DFEAT, HG)),
                  _full_spec((1, HG))],
        out_specs=_row_spec(HG),
        out_shape=jax.ShapeDtypeStruct((NP, HG), jnp.float32),
    )(nodes_p, w_enc, b_enc)


def _tc_scales(ds_pk, dr_pk):
    def body(ds_ref, dr_ref, is1_ref, ir1_ref, is2_ref, ir2_ref):
        ds = ds_ref[0] + ds_ref[1]
        dr = dr_ref[0] + dr_ref[1]
        is1_ref[...] = lax.rsqrt(ds + 1.0)
        ir1_ref[...] = lax.rsqrt(dr + 1.0)
        is2_ref[...] = lax.rsqrt(jnp.maximum(ds, 1.0))
        ir2_ref[...] = lax.rsqrt(jnp.maximum(dr, 1.0))

    hb = NPK // 2
    out = jax.ShapeDtypeStruct((NPK, 128), jnp.float32)
    return pl.pallas_call(
        body,
        grid=(2,),
        in_specs=[_pair_spec(128, hb), _pair_spec(128, hb)],
        out_specs=tuple(_row_spec(128, hb) for _ in range(4)),
        out_shape=(out, out, out, out),
    )(ds_pk, dr_pk)


def _tc_middle(q, w_fc, b_fc, w_mean, b_mean, w_logstd, b_logstd, eps,
               w_dec, b_dec):
    def body(q_ref, wfc_ref, bfc_ref, wm_ref, bm_ref, wl_ref, bl_ref,
             eps_ref, wd_ref, bd_ref, mean_ref, logstd_ref, zh_ref):
        x = q_ref[0] + q_ref[1]
        x = jnp.dot(x, wfc_ref[...], preferred_element_type=jnp.float32,
                    precision=lax.Precision.HIGHEST)
        x = jnp.maximum(x + bfc_ref[...], 0.0)
        mean = jnp.dot(x, wm_ref[...], preferred_element_type=jnp.float32,
                       precision=lax.Precision.HIGHEST) + bm_ref[...]
        logstd = jnp.dot(x, wl_ref[...], preferred_element_type=jnp.float32,
                         precision=lax.Precision.HIGHEST) + bl_ref[...]
        z = mean + jnp.exp(logstd) * eps_ref[...]
        zh = jnp.dot(z, wd_ref[...], preferred_element_type=jnp.float32,
                     precision=lax.Precision.HIGHEST)
        zh_ref[...] = jnp.maximum(zh + bd_ref[...], 0.0)
        mean_ref[...] = mean
        logstd_ref[...] = logstd

    return pl.pallas_call(
        body,
        out_shape=(jax.ShapeDtypeStruct((B, LAT), jnp.float32),
                   jax.ShapeDtypeStruct((B, LAT), jnp.float32),
                   jax.ShapeDtypeStruct((B, N * HG), jnp.float32)),
    )(q, w_fc, b_fc, w_mean, b_mean, w_logstd, b_logstd, eps, w_dec, b_dec)


def _tc_final(q, w_out, b_out):
    def body(q_ref, w_ref, b_ref, o_ref):
        t = q_ref[0] + q_ref[1]
        u = lax.slice(t, (0, 0), (1000, HG))
        v = lax.slice(t, (0, HG), (1000, HG + 1))
        o_ref[...] = jnp.dot(u, w_ref[...],
                             preferred_element_type=jnp.float32,
                             precision=lax.Precision.HIGHEST) + v * b_ref[...]

    return pl.pallas_call(
        body,
        grid=(10,),
        in_specs=[_pair_spec(48, 1000), _full_spec((HG, OUT)),
                  _full_spec((1, OUT))],
        out_specs=_row_spec(OUT, 1000),
        out_shape=jax.ShapeDtypeStruct((NN, OUT), jnp.float32),
    )(q, w_out, b_out)


def kernel(nodes, senders, receivers, eps, w_enc, b_enc, w_fc, b_fc, w_mean,
           b_mean, w_logstd, b_logstd, w_dec, b_dec, w_out, b_out):
    padv = NN + (jnp.arange(PADW, dtype=jnp.int32) % (NP - NN))
    padb = jnp.broadcast_to(padv[None, :], (NW, PADW))
    s_p = jnp.concatenate(
        [senders.reshape(NW, EWR), padb], axis=1).reshape(NW, CH, LW)
    r_p = jnp.concatenate(
        [receivers.reshape(NW, EWR), padb], axis=1).reshape(NW, CH, LW)

    degs_p, degr_p = _sc_degrees(s_p, r_p)

    nodes_p = jnp.pad(nodes, ((0, NP - NN), (0, 0)))
    h = _tc_h(nodes_p, w_enc, b_enc.reshape(1, HG))

    is1, ir1, is2, ir2 = _tc_scales(degs_p.reshape(2, NPK, 128),
                                    degr_p.reshape(2, NPK, 128))

    seg1 = _sc_gcn1(h, is1, ir1, s_p, r_p)

    q1 = seg1[:, :NN].reshape(2, B, N * HG)
    mean, log_std, zh = _tc_middle(
        q1, w_fc, b_fc.reshape(1, HFC), w_mean, b_mean.reshape(1, LAT),
        w_logstd, b_logstd.reshape(1, LAT), eps, w_dec,
        b_dec.reshape(1, N * HG))

    seg2 = _sc_gcn2(zh.reshape(NN, HG), is2, ir2, s_p, r_p)

    outp = _tc_final(seg2, w_out, b_out.reshape(1, OUT))
    return mean, log_std, outp


# speedup vs baseline: 14.3886x; 1.1223x over previous
import functools

import jax
import jax.numpy as jnp
from jax import lax
from jax.experimental import pallas as pl
from jax.experimental.pallas import tpu as pltpu
from jax.experimental.pallas import tpu_sc as plsc

B = 50
N = 200
HG = 32
HFC = 256
LAT = 64
OUT = 128
DFEAT = 128
E = 160000
NN = B * N
NP = 10112
RPT = 632
GPT = RPT // 8
NPK = NP // 8
NW = 32
LW = 128
CH = 40
EP = NW * CH * LW
EWR = E // NW
PADW = CH * LW - EWR
GB = 16
RB = NP // GB

_SLABS = ((0, 128), (128, 128), (256, 128), (384, 128), (512, 120))

_mesh = plsc.VectorSubcoreMesh(core_axis_name="c", subcore_axis_name="s")
_sc_params = pltpu.CompilerParams(use_tc_tiling_on_sc=False)


def _sc_degrees(s_p, r_p):
    out_t = (jax.ShapeDtypeStruct((2, NP, 16), jnp.float32),
             jax.ShapeDtypeStruct((2, NP, 16), jnp.float32))

    @functools.partial(
        pl.kernel, mesh=_mesh, out_type=out_t, compiler_params=_sc_params,
        scratch_types=[
            pltpu.VMEM((CH, LW), jnp.int32),
            pltpu.VMEM((CH, LW), jnp.int32),
            pltpu.VMEM((LW, 16), jnp.float32),
            pltpu.VMEM((RPT, 16), jnp.float32),
            pltpu.VMEM_SHARED((NP, 16), jnp.float32),
            pltpu.VMEM_SHARED((NP, 16), jnp.float32),
            pltpu.SemaphoreType.DMA,
            pltpu.SemaphoreType.DMA,
        ])
    def deg_kernel(s_hbm, r_hbm, ds_out, dr_out, sidx_v, ridx_v, ones_v,
                   zero_v, ds_sh, dr_sh, dsem, rsem):
        c = lax.axis_index("c")
        s = lax.axis_index("s")
        wid = s * 2 + c
        pltpu.sync_copy(s_hbm.at[wid], sidx_v)
        pltpu.sync_copy(r_hbm.at[wid], ridx_v)

        @pl.loop(0, LW)
        def _(i):
            ones_v[i, :] = jnp.ones((16,), jnp.float32)

        @pl.loop(0, RPT)
        def _(i):
            zero_v[i, :] = jnp.zeros((16,), jnp.float32)

        base = s * RPT
        pltpu.sync_copy(zero_v, ds_sh.at[pl.ds(base, RPT)])
        pltpu.sync_copy(zero_v, dr_sh.at[pl.ds(base, RPT)])
        plsc.subcore_barrier()

        @pl.loop(0, CH // 8)
        def _(bb):
            i0 = bb * 8
            for k in range(8):
                pltpu.async_copy(ones_v, ds_sh.at[sidx_v.at[i0 + k]], dsem,
                                 add=True)
                pltpu.async_copy(ones_v, dr_sh.at[ridx_v.at[i0 + k]], rsem,
                                 add=True)
            for k in range(8):
                pltpu.make_async_copy(ones_v, ds_sh.at[sidx_v.at[i0 + k]],
                                      dsem).wait()
                pltpu.make_async_copy(ones_v, dr_sh.at[ridx_v.at[i0 + k]],
                                      rsem).wait()

        plsc.subcore_barrier()
        pltpu.sync_copy(ds_sh.at[pl.ds(base, RPT)],
                        ds_out.at[c, pl.ds(base, RPT)])
        pltpu.sync_copy(dr_sh.at[pl.ds(base, RPT)],
                        dr_out.at[c, pl.ds(base, RPT)])

    return deg_kernel(s_p, r_p)


def _scale_rows(dst, src, sbuf, nlanes):
    @pl.loop(0, GPT)
    def _(g):
        row0 = g * 8
        for k in range(8):
            sv = sbuf[g, pl.ds(16 * k, 16)]
            for h in range(0, nlanes, 16):
                dst[row0 + k, pl.ds(h, 16)] = \
                    src[row0 + k, pl.ds(h, 16)] * sv


def _sc_gcn1(h, is1_pk, ir1_pk, s_p, r_p):
    d = HG

    @functools.partial(
        pl.kernel, mesh=_mesh,
        out_type=jax.ShapeDtypeStruct((2, NP, d), jnp.float32),
        compiler_params=_sc_params,
        scratch_types=[
            pltpu.VMEM((CH, LW), jnp.int32),
            pltpu.VMEM((CH, LW), jnp.int32),
            pltpu.VMEM((4, LW, d), jnp.float32),
            pltpu.VMEM((RPT, d), jnp.float32),
            pltpu.VMEM((GPT, 128), jnp.float32),
            pltpu.VMEM_SHARED((NP, d), jnp.float32),
            pltpu.VMEM_SHARED((NP, d), jnp.float32),
        ] + [pltpu.SemaphoreType.DMA] * 8)
    def k1(h_hbm, is1_hbm, ir1_hbm, s_hbm, r_hbm, out_hbm, sidx_v, ridx_v,
           rows_v, nbuf, sbuf, acc_sh, tab_sh, gs0, gs1, gs2, gs3, ss0, ss1,
           ss2, ss3):
        c = lax.axis_index("c")
        s = lax.axis_index("s")
        wid = s * 2 + c
        base = s * RPT
        pltpu.sync_copy(s_hbm.at[wid], sidx_v)
        pltpu.sync_copy(r_hbm.at[wid], ridx_v)
        pltpu.sync_copy(h_hbm.at[pl.ds(base, RPT)], nbuf)
        pltpu.sync_copy(is1_hbm.at[pl.ds(s * GPT, GPT)], sbuf)
        _scale_rows(nbuf, nbuf, sbuf, d)
        pltpu.sync_copy(nbuf, tab_sh.at[pl.ds(base, RPT)])

        @pl.when(c == 0)
        def _():
            pltpu.sync_copy(nbuf, acc_sh.at[pl.ds(base, RPT)])

        @pl.when(c == 1)
        def _():
            @pl.loop(0, LW)
            def _(i):
                @pl.loop(0, d, step=16)
                def _(l):
                    rows_v[0, i, pl.ds(l, 16)] = jnp.zeros((16,),
                                                           jnp.float32)

            for off, sz in _SLABS:
                pltpu.sync_copy(rows_v.at[0, pl.ds(0, sz)],
                                acc_sh.at[pl.ds(base + off, sz)])

        plsc.subcore_barrier()

        gsems = (gs0, gs1, gs2, gs3)
        ssems = (ss0, ss1, ss2, ss3)
        for b in range(4):
            pltpu.async_copy(tab_sh.at[sidx_v.at[b]], rows_v.at[b], gsems[b])

        @pl.loop(0, CH // 4)
        def _(jj):
            i0 = jj * 4
            for b in range(4):
                i = i0 + b
                pltpu.make_async_copy(tab_sh.at[sidx_v.at[i]], rows_v.at[b],
                                      gsems[b]).wait()
                pltpu.async_copy(rows_v.at[b], acc_sh.at[ridx_v.at[i]],
                                 ssems[b], add=True)
                pb = (b + 3) % 4

                @pl.when((i >= 1) & (i < CH - 3))
                def _():
                    pltpu.make_async_copy(rows_v.at[pb],
                                          acc_sh.at[ridx_v.at[i - 1]],
                                          ssems[pb]).wait()
                    pltpu.async_copy(tab_sh.at[sidx_v.at[i + 3]],
                                     rows_v.at[pb], gsems[pb])

        for b in range(4):
            i = CH - 4 + b
            pltpu.make_async_copy(rows_v.at[b], acc_sh.at[ridx_v.at[i]],
                                  ssems[b]).wait()
        plsc.subcore_barrier()

        pltpu.sync_copy(acc_sh.at[pl.ds(base, RPT)], nbuf)
        pltpu.sync_copy(ir1_hbm.at[pl.ds(s * GPT, GPT)], sbuf)
        _scale_rows(nbuf, nbuf, sbuf, d)
        pltpu.sync_copy(nbuf, out_hbm.at[c, pl.ds(base, RPT)])

    return k1(h, is1_pk, ir1_pk, s_p, r_p)


def _sc_gcn2(zh3, is2_pk, ir2_pk, s_p, r_p):
    d = 48
    HR0, HR1 = 320, RPT - 320
    HG0, HG1 = HR0 // 8, HR1 // 8

    @functools.partial(
        pl.kernel, mesh=_mesh,
        out_type=jax.ShapeDtypeStruct((2, NP, d), jnp.float32),
        compiler_params=_sc_params,
        scratch_types=[
            pltpu.VMEM((CH, LW), jnp.int32),
            pltpu.VMEM((CH, LW), jnp.int32),
            pltpu.VMEM((4, LW, d), jnp.float32),
            pltpu.VMEM((HR0, HG), jnp.float32),
            pltpu.VMEM((HR0, d), jnp.float32),
            pltpu.VMEM((HG0, 128), jnp.float32),
            pltpu.VMEM_SHARED((NP, d), jnp.float32),
            pltpu.VMEM_SHARED((NP, d), jnp.float32),
        ] + [pltpu.SemaphoreType.DMA] * 8)
    def k2(z_hbm, is2_hbm, ir2_hbm, s_hbm, r_hbm, out_hbm, sidx_v, ridx_v,
           rows_v, zbuf, ubuf, sbuf, acc_sh, tab_sh, gs0, gs1, gs2, gs3,
           ss0, ss1, ss2, ss3):
        c = lax.axis_index("c")
        s = lax.axis_index("s")
        wid = s * 2 + c
        base = s * RPT
        pltpu.sync_copy(s_hbm.at[wid], sidx_v)
        pltpu.sync_copy(r_hbm.at[wid], ridx_v)

        if True:
            for p, (hoff, hr, hgrp) in enumerate(((0, HR0, HG0),
                                                  (HR0, HR1, HG1))):
                b0 = base + hoff
                if p == 0:
                    pltpu.sync_copy(z_hbm.at[pl.ds(b0, hr)],
                                    zbuf.at[pl.ds(0, hr)])
                else:
                    @pl.when(s < 15)
                    def _():
                        pltpu.sync_copy(z_hbm.at[pl.ds(b0, hr)],
                                        zbuf.at[pl.ds(0, hr)])

                    @pl.when(s == 15)
                    def _():
                        pltpu.sync_copy(
                            z_hbm.at[pl.ds(b0, NN - 15 * RPT - HR0)],
                            zbuf.at[pl.ds(0, NN - 15 * RPT - HR0)])
                pltpu.sync_copy(
                    is2_hbm.at[pl.ds(s * GPT + hoff // 8, hgrp)],
                    sbuf.at[pl.ds(0, hgrp)])

                @pl.loop(0, hgrp)
                def _(g):
                    row0 = g * 8
                    for k in range(8):
                        sv = sbuf[g, pl.ds(16 * k, 16)]
                        for hh in range(0, HG, 16):
                            ubuf[row0 + k, pl.ds(hh, 16)] = \
                                zbuf[row0 + k, pl.ds(hh, 16)] * sv
                        ubuf[row0 + k, pl.ds(HG, 16)] = sv

                pltpu.sync_copy(ubuf.at[pl.ds(0, hr)],
                                tab_sh.at[pl.ds(b0, hr)])

            @pl.loop(0, LW)
            def _(i):
                @pl.loop(0, d, step=16)
                def _(l):
                    rows_v[0, i, pl.ds(l, 16)] = jnp.zeros((16,),
                                                           jnp.float32)

            for off, sz in _SLABS:
                pltpu.sync_copy(rows_v.at[0, pl.ds(0, sz)],
                                acc_sh.at[pl.ds(base + off, sz)])
            plsc.subcore_barrier()

            gsems = (gs0, gs1, gs2, gs3)
            ssems = (ss0, ss1, ss2, ss3)
            for b in range(4):
                pltpu.async_copy(tab_sh.at[sidx_v.at[b]], rows_v.at[b],
                                 gsems[b])

            @pl.loop(0, CH // 4)
            def _(jj):
                i0 = jj * 4
                for b in range(4):
                    i = i0 + b
                    pltpu.make_async_copy(tab_sh.at[sidx_v.at[i]],
                                          rows_v.at[b], gsems[b]).wait()
                    pltpu.async_copy(rows_v.at[b], acc_sh.at[ridx_v.at[i]],
                                     ssems[b], add=True)
                    pb = (b + 3) % 4

                    @pl.when((i >= 1) & (i < CH - 3))
                    def _():
                        pltpu.make_async_copy(rows_v.at[pb],
                                              acc_sh.at[ridx_v.at[i - 1]],
                                              ssems[pb]).wait()
                        pltpu.async_copy(tab_sh.at[sidx_v.at[i + 3]],
                                         rows_v.at[pb], gsems[pb])

            for b in range(4):
                i = CH - 4 + b
                pltpu.make_async_copy(rows_v.at[b], acc_sh.at[ridx_v.at[i]],
                                      ssems[b]).wait()
            plsc.subcore_barrier()

            for p, (hoff, hr, hgrp) in enumerate(((0, HR0, HG0),
                                                  (HR0, HR1, HG1))):
                b0 = base + hoff
                pltpu.sync_copy(acc_sh.at[pl.ds(b0, hr)],
                                ubuf.at[pl.ds(0, hr)])
                pltpu.sync_copy(
                    ir2_hbm.at[pl.ds(s * GPT + hoff // 8, hgrp)],
                    sbuf.at[pl.ds(0, hgrp)])

                @pl.loop(0, hgrp)
                def _(g):
                    row0 = g * 8
                    for k in range(8):
                        sv = sbuf[g, pl.ds(16 * k, 16)]
                        for hh in range(0, d, 16):
                            ubuf[row0 + k, pl.ds(hh, 16)] = \
                                ubuf[row0 + k, pl.ds(hh, 16)] * sv

                pltpu.sync_copy(ubuf.at[pl.ds(0, hr)],
                                out_hbm.at[c, pl.ds(b0, hr)])

    return k2(zh3, is2_pk, ir2_pk, s_p, r_p)


def _row_spec(width, rows=None):
    return pl.BlockSpec(((rows or RB), width), lambda i: (i, 0))


def _pair_spec(width, rows=None):
    return pl.BlockSpec((2, (rows or RB), width), lambda i: (0, i, 0))


def _full_spec(shape):
    return pl.BlockSpec(shape, lambda i: tuple(0 for _ in shape))


def _tc_h(nodes_p, w_enc, b_enc):
    def body(n_ref, w_ref, b_ref, h_ref):
        h = jnp.dot(n_ref[...], w_ref[...],
                    preferred_element_type=jnp.float32,
                    precision=lax.Precision.HIGHEST)
        h_ref[...] = jnp.maximum(h + b_ref[...], 0.0)

    return pl.pallas_call(
        body,
        grid=(GB,),
        in_specs=[_row_spec(DFEAT), _full_spec((DFEAT, HG)),
                  _full_spec((1, HG))],
        out_specs=_row_spec(HG),
        out_shape=jax.ShapeDtypeStruct((NP, HG), jnp.float32),
    )(nodes_p, w_enc, b_enc)


def _tc_scales(ds_pk, dr_pk):
    def body(ds_ref, dr_ref, is1_ref, ir1_ref, is2_ref, ir2_ref):
        ds = ds_ref[0] + ds_ref[1]
        dr = dr_ref[0] + dr_ref[1]
        is1_ref[...] = lax.rsqrt(ds + 1.0)
        ir1_ref[...] = lax.rsqrt(dr + 1.0)
        is2_ref[...] = lax.rsqrt(jnp.maximum(ds, 1.0))
        ir2_ref[...] = lax.rsqrt(jnp.maximum(dr, 1.0))

    hb = NPK // 2
    out = jax.ShapeDtypeStruct((NPK, 128), jnp.float32)
    return pl.pallas_call(
        body,
        grid=(2,),
        in_specs=[_pair_spec(128, hb), _pair_spec(128, hb)],
        out_specs=tuple(_row_spec(128, hb) for _ in range(4)),
        out_shape=(out, out, out, out),
    )(ds_pk, dr_pk)


def _tc_middle(q, w_fc, b_fc, w_mean, b_mean, w_logstd, b_logstd, eps,
               w_dec, b_dec):
    def body(q_ref, wfc_ref, bfc_ref, wm_ref, bm_ref, wl_ref, bl_ref,
             eps_ref, wd_ref, bd_ref, mean_ref, logstd_ref, zh_ref):
        x = q_ref[0] + q_ref[1]
        x = jnp.dot(x, wfc_ref[...], preferred_element_type=jnp.float32,
                    precision=lax.Precision.HIGHEST)
        x = jnp.maximum(x + bfc_ref[...], 0.0)
        mean = jnp.dot(x, wm_ref[...], preferred_element_type=jnp.float32,
                       precision=lax.Precision.HIGHEST) + bm_ref[...]
        logstd = jnp.dot(x, wl_ref[...], preferred_element_type=jnp.float32,
                         precision=lax.Precision.HIGHEST) + bl_ref[...]
        z = mean + jnp.exp(logstd) * eps_ref[...]
        zh = jnp.dot(z, wd_ref[...], preferred_element_type=jnp.float32,
                     precision=lax.Precision.HIGHEST)
        zh_ref[...] = jnp.maximum(zh + bd_ref[...], 0.0)
        mean_ref[...] = mean
        logstd_ref[...] = logstd

    return pl.pallas_call(
        body,
        out_shape=(jax.ShapeDtypeStruct((B, LAT), jnp.float32),
                   jax.ShapeDtypeStruct((B, LAT), jnp.float32),
                   jax.ShapeDtypeStruct((B, N * HG), jnp.float32)),
    )(q, w_fc, b_fc, w_mean, b_mean, w_logstd, b_logstd, eps, w_dec, b_dec)


def _tc_final(q, w_out, b_out):
    def body(q_ref, w_ref, b_ref, o_ref):
        t = q_ref[0] + q_ref[1]
        u = lax.slice(t, (0, 0), (1000, HG))
        v = lax.slice(t, (0, HG), (1000, HG + 1))
        o_ref[...] = jnp.dot(u, w_ref[...],
                             preferred_element_type=jnp.float32,
                             precision=lax.Precision.HIGHEST) + v * b_ref[...]

    return pl.pallas_call(
        body,
        grid=(10,),
        in_specs=[_pair_spec(48, 1000), _full_spec((HG, OUT)),
                  _full_spec((1, OUT))],
        out_specs=_row_spec(OUT, 1000),
        out_shape=jax.ShapeDtypeStruct((NN, OUT), jnp.float32),
    )(q, w_out, b_out)


def kernel(nodes, senders, receivers, eps, w_enc, b_enc, w_fc, b_fc, w_mean,
           b_mean, w_logstd, b_logstd, w_dec, b_dec, w_out, b_out):
    padv = NN + (jnp.arange(PADW, dtype=jnp.int32) % (NP - NN))
    padb = jnp.broadcast_to(padv[None, :], (NW, PADW))
    s_p = jnp.concatenate(
        [senders.reshape(NW, EWR), padb], axis=1).reshape(NW, CH, LW)
    r_p = jnp.concatenate(
        [receivers.reshape(NW, EWR), padb], axis=1).reshape(NW, CH, LW)

    degs_p, degr_p = _sc_degrees(s_p, r_p)

    nodes_p = jnp.pad(nodes, ((0, NP - NN), (0, 0)))
    h = _tc_h(nodes_p, w_enc, b_enc.reshape(1, HG))

    is1, ir1, is2, ir2 = _tc_scales(degs_p.reshape(2, NPK, 128),
                                    degr_p.reshape(2, NPK, 128))

    seg1 = _sc_gcn1(h, is1, ir1, s_p, r_p)

    q1 = seg1.reshape(2, NP * HG // 128, 128)[:, :NN * HG // 128]
    q1 = q1.reshape(2, B, N * HG)
    mean, log_std, zh = _tc_middle(
        q1, w_fc, b_fc.reshape(1, HFC), w_mean, b_mean.reshape(1, LAT),
        w_logstd, b_logstd.reshape(1, LAT), eps, w_dec,
        b_dec.reshape(1, N * HG))

    seg2 = _sc_gcn2(zh.reshape(NN, HG), is2, ir2, s_p, r_p)

    outp = _tc_final(seg2, w_out, b_out.reshape(1, OUT))
    return mean, log_std, outp


# speedup vs baseline: 15.2925x; 1.0628x over previous
import functools

import jax
import jax.numpy as jnp
from jax import lax
from jax.experimental import pallas as pl
from jax.experimental.pallas import tpu as pltpu
from jax.experimental.pallas import tpu_sc as plsc

B = 50
N = 200
HG = 32
HFC = 256
LAT = 64
OUT = 128
DFEAT = 128
E = 160000
NN = B * N
NP = 10112
RPT = 632
GPT = RPT // 8
NPK = NP // 8
NW = 32
LW = 128
CH = 40
EP = NW * CH * LW
EWR = E // NW
PADW = CH * LW - EWR
GB = 16
RB = NP // GB

_SLABS = ((0, 128), (128, 128), (256, 128), (384, 128), (512, 120))

_mesh = plsc.VectorSubcoreMesh(core_axis_name="c", subcore_axis_name="s")
_sc_params = pltpu.CompilerParams(use_tc_tiling_on_sc=False)


def _sc_degrees(s_p, r_p):
    out_t = (jax.ShapeDtypeStruct((2, NP, 16), jnp.float32),
             jax.ShapeDtypeStruct((2, NP, 16), jnp.float32))

    @functools.partial(
        pl.kernel, mesh=_mesh, out_type=out_t, compiler_params=_sc_params,
        scratch_types=[
            pltpu.VMEM((CH, LW), jnp.int32),
            pltpu.VMEM((CH, LW), jnp.int32),
            pltpu.VMEM((LW, 16), jnp.float32),
            pltpu.VMEM((RPT, 16), jnp.float32),
            pltpu.VMEM_SHARED((NP, 16), jnp.float32),
            pltpu.VMEM_SHARED((NP, 16), jnp.float32),
            pltpu.SemaphoreType.DMA,
            pltpu.SemaphoreType.DMA,
        ])
    def deg_kernel(s_hbm, r_hbm, ds_out, dr_out, sidx_v, ridx_v, ones_v,
                   zero_v, ds_sh, dr_sh, dsem, rsem):
        c = lax.axis_index("c")
        s = lax.axis_index("s")
        wid = s * 2 + c
        pltpu.sync_copy(s_hbm.at[wid], sidx_v)
        pltpu.sync_copy(r_hbm.at[wid], ridx_v)

        @pl.loop(0, LW)
        def _(i):
            ones_v[i, :] = jnp.ones((16,), jnp.float32)

        @pl.loop(0, RPT)
        def _(i):
            zero_v[i, :] = jnp.zeros((16,), jnp.float32)

        base = s * RPT
        pltpu.sync_copy(zero_v, ds_sh.at[pl.ds(base, RPT)])
        pltpu.sync_copy(zero_v, dr_sh.at[pl.ds(base, RPT)])
        plsc.subcore_barrier()

        @pl.loop(0, CH // 8)
        def _(bb):
            i0 = bb * 8
            for k in range(8):
                pltpu.async_copy(ones_v, ds_sh.at[sidx_v.at[i0 + k]], dsem,
                                 add=True)
                pltpu.async_copy(ones_v, dr_sh.at[ridx_v.at[i0 + k]], rsem,
                                 add=True)
            for k in range(8):
                pltpu.make_async_copy(ones_v, ds_sh.at[sidx_v.at[i0 + k]],
                                      dsem).wait()
                pltpu.make_async_copy(ones_v, dr_sh.at[ridx_v.at[i0 + k]],
                                      rsem).wait()

        plsc.subcore_barrier()
        pltpu.sync_copy(ds_sh.at[pl.ds(base, RPT)],
                        ds_out.at[c, pl.ds(base, RPT)])
        pltpu.sync_copy(dr_sh.at[pl.ds(base, RPT)],
                        dr_out.at[c, pl.ds(base, RPT)])

    return deg_kernel(s_p, r_p)


def _scale_rows(dst, src, sbuf, nlanes):
    @plsc.parallel_loop(0, GPT, unroll=2)
    def _(g):
        row0 = g * 8
        for k in range(8):
            sv = sbuf[g, pl.ds(16 * k, 16)]
            for h in range(0, nlanes, 16):
                dst[row0 + k, pl.ds(h, 16)] = \
                    src[row0 + k, pl.ds(h, 16)] * sv


def _sc_gcn1(h, is1_pk, ir1_pk, s_p, r_p):
    d = HG

    @functools.partial(
        pl.kernel, mesh=_mesh,
        out_type=jax.ShapeDtypeStruct((2, NP, d), jnp.float32),
        compiler_params=_sc_params,
        scratch_types=[
            pltpu.VMEM((CH, LW), jnp.int32),
            pltpu.VMEM((CH, LW), jnp.int32),
            pltpu.VMEM((4, LW, d), jnp.float32),
            pltpu.VMEM((RPT, d), jnp.float32),
            pltpu.VMEM((GPT, 128), jnp.float32),
            pltpu.VMEM_SHARED((NP, d), jnp.float32),
            pltpu.VMEM_SHARED((NP, d), jnp.float32),
        ] + [pltpu.SemaphoreType.DMA] * 8)
    def k1(h_hbm, is1_hbm, ir1_hbm, s_hbm, r_hbm, out_hbm, sidx_v, ridx_v,
           rows_v, nbuf, sbuf, acc_sh, tab_sh, gs0, gs1, gs2, gs3, ss0, ss1,
           ss2, ss3):
        c = lax.axis_index("c")
        s = lax.axis_index("s")
        wid = s * 2 + c
        base = s * RPT
        pltpu.sync_copy(s_hbm.at[wid], sidx_v)
        pltpu.sync_copy(r_hbm.at[wid], ridx_v)
        pltpu.sync_copy(h_hbm.at[pl.ds(base, RPT)], nbuf)
        pltpu.sync_copy(is1_hbm.at[pl.ds(s * GPT, GPT)], sbuf)
        _scale_rows(nbuf, nbuf, sbuf, d)
        pltpu.sync_copy(nbuf, tab_sh.at[pl.ds(base, RPT)])

        @pl.when(c == 0)
        def _():
            pltpu.sync_copy(nbuf, acc_sh.at[pl.ds(base, RPT)])

        @pl.when(c == 1)
        def _():
            @pl.loop(0, LW)
            def _(i):
                @pl.loop(0, d, step=16)
                def _(l):
                    rows_v[0, i, pl.ds(l, 16)] = jnp.zeros((16,),
                                                           jnp.float32)

            for off, sz in _SLABS:
                pltpu.sync_copy(rows_v.at[0, pl.ds(0, sz)],
                                acc_sh.at[pl.ds(base + off, sz)])

        plsc.subcore_barrier()

        gsems = (gs0, gs1, gs2, gs3)
        ssems = (ss0, ss1, ss2, ss3)
        for b in range(4):
            pltpu.async_copy(tab_sh.at[sidx_v.at[b]], rows_v.at[b], gsems[b])

        @pl.loop(0, CH // 4)
        def _(jj):
            i0 = jj * 4
            for b in range(4):
                i = i0 + b
                pltpu.make_async_copy(tab_sh.at[sidx_v.at[i]], rows_v.at[b],
                                      gsems[b]).wait()
                pltpu.async_copy(rows_v.at[b], acc_sh.at[ridx_v.at[i]],
                                 ssems[b], add=True)
                pb = (b + 3) % 4

                @pl.when((i >= 1) & (i < CH - 3))
                def _():
                    pltpu.make_async_copy(rows_v.at[pb],
                                          acc_sh.at[ridx_v.at[i - 1]],
                                          ssems[pb]).wait()
                    pltpu.async_copy(tab_sh.at[sidx_v.at[i + 3]],
                                     rows_v.at[pb], gsems[pb])

        for b in range(4):
            i = CH - 4 + b
            pltpu.make_async_copy(rows_v.at[b], acc_sh.at[ridx_v.at[i]],
                                  ssems[b]).wait()
        plsc.subcore_barrier()

        pltpu.sync_copy(acc_sh.at[pl.ds(base, RPT)], nbuf)
        pltpu.sync_copy(ir1_hbm.at[pl.ds(s * GPT, GPT)], sbuf)
        _scale_rows(nbuf, nbuf, sbuf, d)
        pltpu.sync_copy(nbuf, out_hbm.at[c, pl.ds(base, RPT)])

    return k1(h, is1_pk, ir1_pk, s_p, r_p)


def _sc_gcn2(zh3, is2_pk, ir2_pk, s_p, r_p):
    d = 48
    HR0, HR1 = 320, RPT - 320
    HG0, HG1 = HR0 // 8, HR1 // 8

    @functools.partial(
        pl.kernel, mesh=_mesh,
        out_type=jax.ShapeDtypeStruct((2, NP, d), jnp.float32),
        compiler_params=_sc_params,
        scratch_types=[
            pltpu.VMEM((CH, LW), jnp.int32),
            pltpu.VMEM((CH, LW), jnp.int32),
            pltpu.VMEM((4, LW, d), jnp.float32),
            pltpu.VMEM((HR0, HG), jnp.float32),
            pltpu.VMEM((HR0, d), jnp.float32),
            pltpu.VMEM((HG0, 128), jnp.float32),
            pltpu.VMEM_SHARED((NP, d), jnp.float32),
            pltpu.VMEM_SHARED((NP, d), jnp.float32),
        ] + [pltpu.SemaphoreType.DMA] * 8)
    def k2(z_hbm, is2_hbm, ir2_hbm, s_hbm, r_hbm, out_hbm, sidx_v, ridx_v,
           rows_v, zbuf, ubuf, sbuf, acc_sh, tab_sh, gs0, gs1, gs2, gs3,
           ss0, ss1, ss2, ss3):
        c = lax.axis_index("c")
        s = lax.axis_index("s")
        wid = s * 2 + c
        base = s * RPT
        pltpu.sync_copy(s_hbm.at[wid], sidx_v)
        pltpu.sync_copy(r_hbm.at[wid], ridx_v)

        if True:
            for p, (hoff, hr, hgrp) in enumerate(((0, HR0, HG0),
                                                  (HR0, HR1, HG1))):
                b0 = base + hoff
                if p == 0:
                    pltpu.sync_copy(z_hbm.at[pl.ds(b0, hr)],
                                    zbuf.at[pl.ds(0, hr)])
                else:
                    @pl.when(s < 15)
                    def _():
                        pltpu.sync_copy(z_hbm.at[pl.ds(b0, hr)],
                                        zbuf.at[pl.ds(0, hr)])

                    @pl.when(s == 15)
                    def _():
                        pltpu.sync_copy(
                            z_hbm.at[pl.ds(b0, NN - 15 * RPT - HR0)],
                            zbuf.at[pl.ds(0, NN - 15 * RPT - HR0)])
                pltpu.sync_copy(
                    is2_hbm.at[pl.ds(s * GPT + hoff // 8, hgrp)],
                    sbuf.at[pl.ds(0, hgrp)])

                @plsc.parallel_loop(0, hgrp, unroll=2)
                def _(g):
                    row0 = g * 8
                    for k in range(8):
                        sv = sbuf[g, pl.ds(16 * k, 16)]
                        for hh in range(0, HG, 16):
                            ubuf[row0 + k, pl.ds(hh, 16)] = \
                                zbuf[row0 + k, pl.ds(hh, 16)] * sv
                        ubuf[row0 + k, pl.ds(HG, 16)] = sv

                pltpu.sync_copy(ubuf.at[pl.ds(0, hr)],
                                tab_sh.at[pl.ds(b0, hr)])

            @pl.loop(0, LW)
            def _(i):
                @pl.loop(0, d, step=16)
                def _(l):
                    rows_v[0, i, pl.ds(l, 16)] = jnp.zeros((16,),
                                                           jnp.float32)

            for off, sz in _SLABS:
                pltpu.sync_copy(rows_v.at[0, pl.ds(0, sz)],
                                acc_sh.at[pl.ds(base + off, sz)])
            plsc.subcore_barrier()

            gsems = (gs0, gs1, gs2, gs3)
            ssems = (ss0, ss1, ss2, ss3)
            for b in range(4):
                pltpu.async_copy(tab_sh.at[sidx_v.at[b]], rows_v.at[b],
                                 gsems[b])

            @pl.loop(0, CH // 4)
            def _(jj):
                i0 = jj * 4
                for b in range(4):
                    i = i0 + b
                    pltpu.make_async_copy(tab_sh.at[sidx_v.at[i]],
                                          rows_v.at[b], gsems[b]).wait()
                    pltpu.async_copy(rows_v.at[b], acc_sh.at[ridx_v.at[i]],
                                     ssems[b], add=True)
                    pb = (b + 3) % 4

                    @pl.when((i >= 1) & (i < CH - 3))
                    def _():
                        pltpu.make_async_copy(rows_v.at[pb],
                                              acc_sh.at[ridx_v.at[i - 1]],
                                              ssems[pb]).wait()
                        pltpu.async_copy(tab_sh.at[sidx_v.at[i + 3]],
                                         rows_v.at[pb], gsems[pb])

            for b in range(4):
                i = CH - 4 + b
                pltpu.make_async_copy(rows_v.at[b], acc_sh.at[ridx_v.at[i]],
                                      ssems[b]).wait()
            plsc.subcore_barrier()

            for p, (hoff, hr, hgrp) in enumerate(((0, HR0, HG0),
                                                  (HR0, HR1, HG1))):
                b0 = base + hoff
                pltpu.sync_copy(acc_sh.at[pl.ds(b0, hr)],
                                ubuf.at[pl.ds(0, hr)])
                pltpu.sync_copy(
                    ir2_hbm.at[pl.ds(s * GPT + hoff // 8, hgrp)],
                    sbuf.at[pl.ds(0, hgrp)])

                @plsc.parallel_loop(0, hgrp, unroll=2)
                def _(g):
                    row0 = g * 8
                    for k in range(8):
                        sv = sbuf[g, pl.ds(16 * k, 16)]
                        for hh in range(0, d, 16):
                            ubuf[row0 + k, pl.ds(hh, 16)] = \
                                ubuf[row0 + k, pl.ds(hh, 16)] * sv

                pltpu.sync_copy(ubuf.at[pl.ds(0, hr)],
                                out_hbm.at[c, pl.ds(b0, hr)])

    return k2(zh3, is2_pk, ir2_pk, s_p, r_p)


def _row_spec(width, rows=None):
    return pl.BlockSpec(((rows or RB), width), lambda i: (i, 0))


def _pair_spec(width, rows=None):
    return pl.BlockSpec((2, (rows or RB), width), lambda i: (0, i, 0))


def _full_spec(shape):
    return pl.BlockSpec(shape, lambda i: tuple(0 for _ in shape))


def _tc_h(nodes_p, w_enc, b_enc):
    def body(n_ref, w_ref, b_ref, h_ref):
        h = jnp.dot(n_ref[...], w_ref[...],
                    preferred_element_type=jnp.float32,
                    precision=lax.Precision.HIGHEST)
        h_ref[...] = jnp.maximum(h + b_ref[...], 0.0)

    return pl.pallas_call(
        body,
        grid=(GB,),
        in_specs=[_row_spec(DFEAT), _full_spec((DFEAT, HG)),
                  _full_spec((1, HG))],
        out_specs=_row_spec(HG),
        out_shape=jax.ShapeDtypeStruct((NP, HG), jnp.float32),
    )(nodes_p, w_enc, b_enc)


def _tc_scales(ds_pk, dr_pk):
    def body(ds_ref, dr_ref, is1_ref, ir1_ref, is2_ref, ir2_ref):
        ds = ds_ref[0] + ds_ref[1]
        dr = dr_ref[0] + dr_ref[1]
        is1_ref[...] = lax.rsqrt(ds + 1.0)
        ir1_ref[...] = lax.rsqrt(dr + 1.0)
        is2_ref[...] = lax.rsqrt(jnp.maximum(ds, 1.0))
        ir2_ref[...] = lax.rsqrt(jnp.maximum(dr, 1.0))

    hb = NPK // 2
    out = jax.ShapeDtypeStruct((NPK, 128), jnp.float32)
    return pl.pallas_call(
        body,
        grid=(2,),
        in_specs=[_pair_spec(128, hb), _pair_spec(128, hb)],
        out_specs=tuple(_row_spec(128, hb) for _ in range(4)),
        out_shape=(out, out, out, out),
    )(ds_pk, dr_pk)


def _tc_middle(q, w_fc, b_fc, w_mean, b_mean, w_logstd, b_logstd, eps,
               w_dec, b_dec):
    def body(q_ref, wfc_ref, bfc_ref, wm_ref, bm_ref, wl_ref, bl_ref,
             eps_ref, wd_ref, bd_ref, mean_ref, logstd_ref, zh_ref):
        x = q_ref[0] + q_ref[1]
        x = jnp.dot(x, wfc_ref[...], preferred_element_type=jnp.float32,
                    precision=lax.Precision.HIGHEST)
        x = jnp.maximum(x + bfc_ref[...], 0.0)
        mean = jnp.dot(x, wm_ref[...], preferred_element_type=jnp.float32,
                       precision=lax.Precision.HIGHEST) + bm_ref[...]
        logstd = jnp.dot(x, wl_ref[...], preferred_element_type=jnp.float32,
                         precision=lax.Precision.HIGHEST) + bl_ref[...]
        z = mean + jnp.exp(logstd) * eps_ref[...]
        zh = jnp.dot(z, wd_ref[...], preferred_element_type=jnp.float32,
                     precision=lax.Precision.HIGHEST)
        zh_ref[...] = jnp.maximum(zh + bd_ref[...], 0.0)
        mean_ref[...] = mean
        logstd_ref[...] = logstd

    return pl.pallas_call(
        body,
        out_shape=(jax.ShapeDtypeStruct((B, LAT), jnp.float32),
                   jax.ShapeDtypeStruct((B, LAT), jnp.float32),
                   jax.ShapeDtypeStruct((B, N * HG), jnp.float32)),
    )(q, w_fc, b_fc, w_mean, b_mean, w_logstd, b_logstd, eps, w_dec, b_dec)


def _tc_final(q, w_out, b_out):
    def body(q_ref, w_ref, b_ref, o_ref):
        t = q_ref[0] + q_ref[1]
        u = lax.slice(t, (0, 0), (1000, HG))
        v = lax.slice(t, (0, HG), (1000, HG + 1))
        o_ref[...] = jnp.dot(u, w_ref[...],
                             preferred_element_type=jnp.float32,
                             precision=lax.Precision.HIGHEST) + v * b_ref[...]

    return pl.pallas_call(
        body,
        grid=(10,),
        in_specs=[_pair_spec(48, 1000), _full_spec((HG, OUT)),
                  _full_spec((1, OUT))],
        out_specs=_row_spec(OUT, 1000),
        out_shape=jax.ShapeDtypeStruct((NN, OUT), jnp.float32),
    )(q, w_out, b_out)


def kernel(nodes, senders, receivers, eps, w_enc, b_enc, w_fc, b_fc, w_mean,
           b_mean, w_logstd, b_logstd, w_dec, b_dec, w_out, b_out):
    padv = NN + (jnp.arange(PADW, dtype=jnp.int32) % (NP - NN))
    padb = jnp.broadcast_to(padv[None, :], (NW, PADW))
    s_p = jnp.concatenate(
        [senders.reshape(NW, EWR), padb], axis=1).reshape(NW, CH, LW)
    r_p = jnp.concatenate(
        [receivers.reshape(NW, EWR), padb], axis=1).reshape(NW, CH, LW)

    degs_p, degr_p = _sc_degrees(s_p, r_p)

    nodes_p = jnp.pad(nodes, ((0, NP - NN), (0, 0)))
    h = _tc_h(nodes_p, w_enc, b_enc.reshape(1, HG))

    is1, ir1, is2, ir2 = _tc_scales(degs_p.reshape(2, NPK, 128),
                                    degr_p.reshape(2, NPK, 128))

    seg1 = _sc_gcn1(h, is1, ir1, s_p, r_p)

    q1 = seg1.reshape(2, NP * HG // 128, 128)[:, :NN * HG // 128]
    q1 = q1.reshape(2, B, N * HG)
    mean, log_std, zh = _tc_middle(
        q1, w_fc, b_fc.reshape(1, HFC), w_mean, b_mean.reshape(1, LAT),
        w_logstd, b_logstd.reshape(1, LAT), eps, w_dec,
        b_dec.reshape(1, N * HG))

    seg2 = _sc_gcn2(zh.reshape(NN, HG), is2, ir2, s_p, r_p)

    outp = _tc_final(seg2, w_out, b_out.reshape(1, OUT))
    return mean, log_std, outp


# speedup vs baseline: 15.3418x; 1.0032x over previous
import functools

import jax
import jax.numpy as jnp
from jax import lax
from jax.experimental import pallas as pl
from jax.experimental.pallas import tpu as pltpu
from jax.experimental.pallas import tpu_sc as plsc

B = 50
N = 200
HG = 32
HFC = 256
LAT = 64
OUT = 128
DFEAT = 128
E = 160000
NN = B * N
NP = 10112
RPT = 632
GPT = RPT // 8
NPK = NP // 8
NW = 32
LW = 128
CH = 40
EP = NW * CH * LW
EWR = E // NW
PADW = CH * LW - EWR
GB = 16
RB = NP // GB

_SLABS = ((0, 128), (128, 128), (256, 128), (384, 128), (512, 120))

_mesh = plsc.VectorSubcoreMesh(core_axis_name="c", subcore_axis_name="s")
_sc_params = pltpu.CompilerParams(use_tc_tiling_on_sc=False)


def _sc_degrees(s_p, r_p):
    out_t = (jax.ShapeDtypeStruct((2, NP, 16), jnp.float32),
             jax.ShapeDtypeStruct((2, NP, 16), jnp.float32))

    @functools.partial(
        pl.kernel, mesh=_mesh, out_type=out_t, compiler_params=_sc_params,
        scratch_types=[
            pltpu.VMEM((CH, LW), jnp.int32),
            pltpu.VMEM((CH, LW), jnp.int32),
            pltpu.VMEM((LW, 16), jnp.float32),
            pltpu.VMEM((RPT, 16), jnp.float32),
            pltpu.VMEM_SHARED((NP, 16), jnp.float32),
            pltpu.VMEM_SHARED((NP, 16), jnp.float32),
            pltpu.SemaphoreType.DMA,
            pltpu.SemaphoreType.DMA,
        ])
    def deg_kernel(s_hbm, r_hbm, ds_out, dr_out, sidx_v, ridx_v, ones_v,
                   zero_v, ds_sh, dr_sh, dsem, rsem):
        c = lax.axis_index("c")
        s = lax.axis_index("s")
        wid = s * 2 + c
        pltpu.sync_copy(s_hbm.at[wid], sidx_v)
        pltpu.sync_copy(r_hbm.at[wid], ridx_v)

        @plsc.parallel_loop(0, LW, unroll=4)
        def _(i):
            ones_v[i, :] = jnp.ones((16,), jnp.float32)

        @plsc.parallel_loop(0, RPT, unroll=4)
        def _(i):
            zero_v[i, :] = jnp.zeros((16,), jnp.float32)

        base = s * RPT
        pltpu.sync_copy(zero_v, ds_sh.at[pl.ds(base, RPT)])
        pltpu.sync_copy(zero_v, dr_sh.at[pl.ds(base, RPT)])
        plsc.subcore_barrier()

        @pl.loop(0, CH // 8)
        def _(bb):
            i0 = bb * 8
            for k in range(8):
                pltpu.async_copy(ones_v, ds_sh.at[sidx_v.at[i0 + k]], dsem,
                                 add=True)
                pltpu.async_copy(ones_v, dr_sh.at[ridx_v.at[i0 + k]], rsem,
                                 add=True)
            for k in range(8):
                pltpu.make_async_copy(ones_v, ds_sh.at[sidx_v.at[i0 + k]],
                                      dsem).wait()
                pltpu.make_async_copy(ones_v, dr_sh.at[ridx_v.at[i0 + k]],
                                      rsem).wait()

        plsc.subcore_barrier()
        pltpu.sync_copy(ds_sh.at[pl.ds(base, RPT)],
                        ds_out.at[c, pl.ds(base, RPT)])
        pltpu.sync_copy(dr_sh.at[pl.ds(base, RPT)],
                        dr_out.at[c, pl.ds(base, RPT)])

    return deg_kernel(s_p, r_p)


def _scale_rows(dst, src, sbuf, nlanes):
    @plsc.parallel_loop(0, GPT, unroll=4)
    def _(g):
        row0 = g * 8
        for k in range(8):
            sv = sbuf[g, pl.ds(16 * k, 16)]
            for h in range(0, nlanes, 16):
                dst[row0 + k, pl.ds(h, 16)] = \
                    src[row0 + k, pl.ds(h, 16)] * sv


def _sc_gcn1(h, is1_pk, ir1_pk, s_p, r_p):
    d = HG

    @functools.partial(
        pl.kernel, mesh=_mesh,
        out_type=jax.ShapeDtypeStruct((2, NP, d), jnp.float32),
        compiler_params=_sc_params,
        scratch_types=[
            pltpu.VMEM((CH, LW), jnp.int32),
            pltpu.VMEM((CH, LW), jnp.int32),
            pltpu.VMEM((4, LW, d), jnp.float32),
            pltpu.VMEM((RPT, d), jnp.float32),
            pltpu.VMEM((GPT, 128), jnp.float32),
            pltpu.VMEM_SHARED((NP, d), jnp.float32),
            pltpu.VMEM_SHARED((NP, d), jnp.float32),
        ] + [pltpu.SemaphoreType.DMA] * 8)
    def k1(h_hbm, is1_hbm, ir1_hbm, s_hbm, r_hbm, out_hbm, sidx_v, ridx_v,
           rows_v, nbuf, sbuf, acc_sh, tab_sh, gs0, gs1, gs2, gs3, ss0, ss1,
           ss2, ss3):
        c = lax.axis_index("c")
        s = lax.axis_index("s")
        wid = s * 2 + c
        base = s * RPT
        pltpu.sync_copy(s_hbm.at[wid], sidx_v)
        pltpu.sync_copy(r_hbm.at[wid], ridx_v)
        pltpu.sync_copy(h_hbm.at[pl.ds(base, RPT)], nbuf)
        pltpu.sync_copy(is1_hbm.at[pl.ds(s * GPT, GPT)], sbuf)
        _scale_rows(nbuf, nbuf, sbuf, d)
        pltpu.sync_copy(nbuf, tab_sh.at[pl.ds(base, RPT)])

        @pl.when(c == 0)
        def _():
            pltpu.sync_copy(nbuf, acc_sh.at[pl.ds(base, RPT)])

        @pl.when(c == 1)
        def _():
            @plsc.parallel_loop(0, LW, unroll=4)
            def _(i):
                for l in range(0, d, 16):
                    rows_v[0, i, pl.ds(l, 16)] = jnp.zeros((16,),
                                                           jnp.float32)

            for off, sz in _SLABS:
                pltpu.sync_copy(rows_v.at[0, pl.ds(0, sz)],
                                acc_sh.at[pl.ds(base + off, sz)])

        plsc.subcore_barrier()

        gsems = (gs0, gs1, gs2, gs3)
        ssems = (ss0, ss1, ss2, ss3)
        for b in range(4):
            pltpu.async_copy(tab_sh.at[sidx_v.at[b]], rows_v.at[b], gsems[b])

        @pl.loop(0, CH // 4)
        def _(jj):
            i0 = jj * 4
            for b in range(4):
                i = i0 + b
                pltpu.make_async_copy(tab_sh.at[sidx_v.at[i]], rows_v.at[b],
                                      gsems[b]).wait()
                pltpu.async_copy(rows_v.at[b], acc_sh.at[ridx_v.at[i]],
                                 ssems[b], add=True)
                pb = (b + 3) % 4

                @pl.when((i >= 1) & (i < CH - 3))
                def _():
                    pltpu.make_async_copy(rows_v.at[pb],
                                          acc_sh.at[ridx_v.at[i - 1]],
                                          ssems[pb]).wait()
                    pltpu.async_copy(tab_sh.at[sidx_v.at[i + 3]],
                                     rows_v.at[pb], gsems[pb])

        for b in range(4):
            i = CH - 4 + b
            pltpu.make_async_copy(rows_v.at[b], acc_sh.at[ridx_v.at[i]],
                                  ssems[b]).wait()
        plsc.subcore_barrier()

        pltpu.sync_copy(acc_sh.at[pl.ds(base, RPT)], nbuf)
        pltpu.sync_copy(ir1_hbm.at[pl.ds(s * GPT, GPT)], sbuf)
        _scale_rows(nbuf, nbuf, sbuf, d)
        pltpu.sync_copy(nbuf, out_hbm.at[c, pl.ds(base, RPT)])

    return k1(h, is1_pk, ir1_pk, s_p, r_p)


def _sc_gcn2(zh3, is2_pk, ir2_pk, s_p, r_p):
    d = 48
    HR0, HR1 = 320, RPT - 320
    HG0, HG1 = HR0 // 8, HR1 // 8

    @functools.partial(
        pl.kernel, mesh=_mesh,
        out_type=jax.ShapeDtypeStruct((2, NP, d), jnp.float32),
        compiler_params=_sc_params,
        scratch_types=[
            pltpu.VMEM((CH, LW), jnp.int32),
            pltpu.VMEM((CH, LW), jnp.int32),
            pltpu.VMEM((4, LW, d), jnp.float32),
            pltpu.VMEM((HR0, HG), jnp.float32),
            pltpu.VMEM((HR0, d), jnp.float32),
            pltpu.VMEM((HG0, 128), jnp.float32),
            pltpu.VMEM_SHARED((NP, d), jnp.float32),
            pltpu.VMEM_SHARED((NP, d), jnp.float32),
        ] + [pltpu.SemaphoreType.DMA] * 8)
    def k2(z_hbm, is2_hbm, ir2_hbm, s_hbm, r_hbm, out_hbm, sidx_v, ridx_v,
           rows_v, zbuf, ubuf, sbuf, acc_sh, tab_sh, gs0, gs1, gs2, gs3,
           ss0, ss1, ss2, ss3):
        c = lax.axis_index("c")
        s = lax.axis_index("s")
        wid = s * 2 + c
        base = s * RPT
        pltpu.sync_copy(s_hbm.at[wid], sidx_v)
        pltpu.sync_copy(r_hbm.at[wid], ridx_v)

        if True:
            for p, (hoff, hr, hgrp) in enumerate(((0, HR0, HG0),
                                                  (HR0, HR1, HG1))):
                b0 = base + hoff
                if p == 0:
                    pltpu.sync_copy(z_hbm.at[pl.ds(b0, hr)],
                                    zbuf.at[pl.ds(0, hr)])
                else:
                    @pl.when(s < 15)
                    def _():
                        pltpu.sync_copy(z_hbm.at[pl.ds(b0, hr)],
                                        zbuf.at[pl.ds(0, hr)])

                    @pl.when(s == 15)
                    def _():
                        pltpu.sync_copy(
                            z_hbm.at[pl.ds(b0, NN - 15 * RPT - HR0)],
                            zbuf.at[pl.ds(0, NN - 15 * RPT - HR0)])
                pltpu.sync_copy(
                    is2_hbm.at[pl.ds(s * GPT + hoff // 8, hgrp)],
                    sbuf.at[pl.ds(0, hgrp)])

                @plsc.parallel_loop(0, hgrp, unroll=4)
                def _(g):
                    row0 = g * 8
                    for k in range(8):
                        sv = sbuf[g, pl.ds(16 * k, 16)]
                        for hh in range(0, HG, 16):
                            ubuf[row0 + k, pl.ds(hh, 16)] = \
                                zbuf[row0 + k, pl.ds(hh, 16)] * sv
                        ubuf[row0 + k, pl.ds(HG, 16)] = sv

                pltpu.sync_copy(ubuf.at[pl.ds(0, hr)],
                                tab_sh.at[pl.ds(b0, hr)])

            @plsc.parallel_loop(0, LW, unroll=4)
            def _(i):
                for l in range(0, d, 16):
                    rows_v[0, i, pl.ds(l, 16)] = jnp.zeros((16,),
                                                           jnp.float32)

            for off, sz in _SLABS:
                pltpu.sync_copy(rows_v.at[0, pl.ds(0, sz)],
                                acc_sh.at[pl.ds(base + off, sz)])
            plsc.subcore_barrier()

            gsems = (gs0, gs1, gs2, gs3)
            ssems = (ss0, ss1, ss2, ss3)
            for b in range(4):
                pltpu.async_copy(tab_sh.at[sidx_v.at[b]], rows_v.at[b],
                                 gsems[b])

            @pl.loop(0, CH // 4)
            def _(jj):
                i0 = jj * 4
                for b in range(4):
                    i = i0 + b
                    pltpu.make_async_copy(tab_sh.at[sidx_v.at[i]],
                                          rows_v.at[b], gsems[b]).wait()
                    pltpu.async_copy(rows_v.at[b], acc_sh.at[ridx_v.at[i]],
                                     ssems[b], add=True)
                    pb = (b + 3) % 4

                    @pl.when((i >= 1) & (i < CH - 3))
                    def _():
                        pltpu.make_async_copy(rows_v.at[pb],
                                              acc_sh.at[ridx_v.at[i - 1]],
                                              ssems[pb]).wait()
                        pltpu.async_copy(tab_sh.at[sidx_v.at[i + 3]],
                                         rows_v.at[pb], gsems[pb])

            for b in range(4):
                i = CH - 4 + b
                pltpu.make_async_copy(rows_v.at[b], acc_sh.at[ridx_v.at[i]],
                                      ssems[b]).wait()
            plsc.subcore_barrier()

            for p, (hoff, hr, hgrp) in enumerate(((0, HR0, HG0),
                                                  (HR0, HR1, HG1))):
                b0 = base + hoff
                pltpu.sync_copy(acc_sh.at[pl.ds(b0, hr)],
                                ubuf.at[pl.ds(0, hr)])
                pltpu.sync_copy(
                    ir2_hbm.at[pl.ds(s * GPT + hoff // 8, hgrp)],
                    sbuf.at[pl.ds(0, hgrp)])

                @plsc.parallel_loop(0, hgrp, unroll=4)
                def _(g):
                    row0 = g * 8
                    for k in range(8):
                        sv = sbuf[g, pl.ds(16 * k, 16)]
                        for hh in range(0, d, 16):
                            ubuf[row0 + k, pl.ds(hh, 16)] = \
                                ubuf[row0 + k, pl.ds(hh, 16)] * sv

                pltpu.sync_copy(ubuf.at[pl.ds(0, hr)],
                                out_hbm.at[c, pl.ds(b0, hr)])

    return k2(zh3, is2_pk, ir2_pk, s_p, r_p)


def _row_spec(width, rows=None):
    return pl.BlockSpec(((rows or RB), width), lambda i: (i, 0))


def _pair_spec(width, rows=None):
    return pl.BlockSpec((2, (rows or RB), width), lambda i: (0, i, 0))


def _full_spec(shape):
    return pl.BlockSpec(shape, lambda i: tuple(0 for _ in shape))


def _tc_h(nodes_p, w_enc, b_enc):
    def body(n_ref, w_ref, b_ref, h_ref):
        h = jnp.dot(n_ref[...], w_ref[...],
                    preferred_element_type=jnp.float32,
                    precision=lax.Precision.HIGHEST)
        h_ref[...] = jnp.maximum(h + b_ref[...], 0.0)

    return pl.pallas_call(
        body,
        grid=(GB,),
        in_specs=[_row_spec(DFEAT), _full_spec((DFEAT, HG)),
                  _full_spec((1, HG))],
        out_specs=_row_spec(HG),
        out_shape=jax.ShapeDtypeStruct((NP, HG), jnp.float32),
    )(nodes_p, w_enc, b_enc)


def _tc_scales(ds_pk, dr_pk):
    def body(ds_ref, dr_ref, is1_ref, ir1_ref, is2_ref, ir2_ref):
        ds = ds_ref[0] + ds_ref[1]
        dr = dr_ref[0] + dr_ref[1]
        is1_ref[...] = lax.rsqrt(ds + 1.0)
        ir1_ref[...] = lax.rsqrt(dr + 1.0)
        is2_ref[...] = lax.rsqrt(jnp.maximum(ds, 1.0))
        ir2_ref[...] = lax.rsqrt(jnp.maximum(dr, 1.0))

    hb = NPK // 2
    out = jax.ShapeDtypeStruct((NPK, 128), jnp.float32)
    return pl.pallas_call(
        body,
        grid=(2,),
        in_specs=[_pair_spec(128, hb), _pair_spec(128, hb)],
        out_specs=tuple(_row_spec(128, hb) for _ in range(4)),
        out_shape=(out, out, out, out),
    )(ds_pk, dr_pk)


def _tc_middle(q, w_fc, b_fc, w_mean, b_mean, w_logstd, b_logstd, eps,
               w_dec, b_dec):
    def body(q_ref, wfc_ref, bfc_ref, wm_ref, bm_ref, wl_ref, bl_ref,
             eps_ref, wd_ref, bd_ref, mean_ref, logstd_ref, zh_ref):
        x = q_ref[0] + q_ref[1]
        x = jnp.dot(x, wfc_ref[...], preferred_element_type=jnp.float32,
                    precision=lax.Precision.HIGHEST)
        x = jnp.maximum(x + bfc_ref[...], 0.0)
        mean = jnp.dot(x, wm_ref[...], preferred_element_type=jnp.float32,
                       precision=lax.Precision.HIGHEST) + bm_ref[...]
        logstd = jnp.dot(x, wl_ref[...], preferred_element_type=jnp.float32,
                         precision=lax.Precision.HIGHEST) + bl_ref[...]
        z = mean + jnp.exp(logstd) * eps_ref[...]
        zh = jnp.dot(z, wd_ref[...], preferred_element_type=jnp.float32,
                     precision=lax.Precision.HIGHEST)
        zh_ref[...] = jnp.maximum(zh + bd_ref[...], 0.0)
        mean_ref[...] = mean
        logstd_ref[...] = logstd

    return pl.pallas_call(
        body,
        out_shape=(jax.ShapeDtypeStruct((B, LAT), jnp.float32),
                   jax.ShapeDtypeStruct((B, LAT), jnp.float32),
                   jax.ShapeDtypeStruct((B, N * HG), jnp.float32)),
    )(q, w_fc, b_fc, w_mean, b_mean, w_logstd, b_logstd, eps, w_dec, b_dec)


def _tc_final(q, w_out, b_out):
    def body(q_ref, w_ref, b_ref, o_ref):
        t = q_ref[0] + q_ref[1]
        u = lax.slice(t, (0, 0), (1000, HG))
        v = lax.slice(t, (0, HG), (1000, HG + 1))
        o_ref[...] = jnp.dot(u, w_ref[...],
                             preferred_element_type=jnp.float32,
                             precision=lax.Precision.HIGHEST) + v * b_ref[...]

    return pl.pallas_call(
        body,
        grid=(10,),
        in_specs=[_pair_spec(48, 1000), _full_spec((HG, OUT)),
                  _full_spec((1, OUT))],
        out_specs=_row_spec(OUT, 1000),
        out_shape=jax.ShapeDtypeStruct((NN, OUT), jnp.float32),
    )(q, w_out, b_out)


def kernel(nodes, senders, receivers, eps, w_enc, b_enc, w_fc, b_fc, w_mean,
           b_mean, w_logstd, b_logstd, w_dec, b_dec, w_out, b_out):
    padv = NN + (jnp.arange(PADW, dtype=jnp.int32) % (NP - NN))
    padb = jnp.broadcast_to(padv[None, :], (NW, PADW))
    s_p = jnp.concatenate(
        [senders.reshape(NW, EWR), padb], axis=1).reshape(NW, CH, LW)
    r_p = jnp.concatenate(
        [receivers.reshape(NW, EWR), padb], axis=1).reshape(NW, CH, LW)

    degs_p, degr_p = _sc_degrees(s_p, r_p)

    nodes_p = jnp.pad(nodes, ((0, NP - NN), (0, 0)))
    h = _tc_h(nodes_p, w_enc, b_enc.reshape(1, HG))

    is1, ir1, is2, ir2 = _tc_scales(degs_p.reshape(2, NPK, 128),
                                    degr_p.reshape(2, NPK, 128))

    seg1 = _sc_gcn1(h, is1, ir1, s_p, r_p)

    q1 = seg1.reshape(2, NP * HG // 128, 128)[:, :NN * HG // 128]
    q1 = q1.reshape(2, B, N * HG)
    mean, log_std, zh = _tc_middle(
        q1, w_fc, b_fc.reshape(1, HFC), w_mean, b_mean.reshape(1, LAT),
        w_logstd, b_logstd.reshape(1, LAT), eps, w_dec,
        b_dec.reshape(1, N * HG))

    seg2 = _sc_gcn2(zh.reshape(NN, HG), is2, ir2, s_p, r_p)

    outp = _tc_final(seg2, w_out, b_out.reshape(1, OUT))
    return mean, log_std, outp


# speedup vs baseline: 15.3562x; 1.0009x over previous
import functools

import jax
import jax.numpy as jnp
from jax import lax
from jax.experimental import pallas as pl
from jax.experimental.pallas import tpu as pltpu
from jax.experimental.pallas import tpu_sc as plsc

B = 50
N = 200
HG = 32
HFC = 256
LAT = 64
OUT = 128
DFEAT = 128
E = 160000
NN = B * N
NP = 10112
RPT = 632
GPT = RPT // 8
NPK = NP // 8
NW = 32
LW = 128
CH = 40
EP = NW * CH * LW
EWR = E // NW
PADW = CH * LW - EWR
GB = 16
RB = NP // GB

_SLABS = ((0, 128), (128, 128), (256, 128), (384, 128), (512, 120))

def _mesh():
    return plsc.VectorSubcoreMesh(core_axis_name="c", subcore_axis_name="s")


_sc_params = pltpu.CompilerParams(use_tc_tiling_on_sc=False)


def _sc_degrees(s_p, r_p):
    out_t = (jax.ShapeDtypeStruct((2, NP, 16), jnp.float32),
             jax.ShapeDtypeStruct((2, NP, 16), jnp.float32))

    @functools.partial(
        pl.kernel, mesh=_mesh(), out_type=out_t, compiler_params=_sc_params,
        scratch_types=[
            pltpu.VMEM((CH, LW), jnp.int32),
            pltpu.VMEM((CH, LW), jnp.int32),
            pltpu.VMEM((LW, 16), jnp.float32),
            pltpu.VMEM((RPT, 16), jnp.float32),
            pltpu.VMEM_SHARED((NP, 16), jnp.float32),
            pltpu.VMEM_SHARED((NP, 16), jnp.float32),
            pltpu.SemaphoreType.DMA,
            pltpu.SemaphoreType.DMA,
        ])
    def deg_kernel(s_hbm, r_hbm, ds_out, dr_out, sidx_v, ridx_v, ones_v,
                   zero_v, ds_sh, dr_sh, dsem, rsem):
        c = lax.axis_index("c")
        s = lax.axis_index("s")
        wid = s * 2 + c
        pltpu.sync_copy(s_hbm.at[wid], sidx_v)
        pltpu.sync_copy(r_hbm.at[wid], ridx_v)

        @plsc.parallel_loop(0, LW, unroll=4)
        def _(i):
            ones_v[i, :] = jnp.ones((16,), jnp.float32)

        @plsc.parallel_loop(0, RPT, unroll=4)
        def _(i):
            zero_v[i, :] = jnp.zeros((16,), jnp.float32)

        base = s * RPT
        pltpu.sync_copy(zero_v, ds_sh.at[pl.ds(base, RPT)])
        pltpu.sync_copy(zero_v, dr_sh.at[pl.ds(base, RPT)])
        plsc.subcore_barrier()

        @pl.loop(0, CH // 8)
        def _(bb):
            i0 = bb * 8
            for k in range(8):
                pltpu.async_copy(ones_v, ds_sh.at[sidx_v.at[i0 + k]], dsem,
                                 add=True)
                pltpu.async_copy(ones_v, dr_sh.at[ridx_v.at[i0 + k]], rsem,
                                 add=True)
            for k in range(8):
                pltpu.make_async_copy(ones_v, ds_sh.at[sidx_v.at[i0 + k]],
                                      dsem).wait()
                pltpu.make_async_copy(ones_v, dr_sh.at[ridx_v.at[i0 + k]],
                                      rsem).wait()

        plsc.subcore_barrier()
        pltpu.sync_copy(ds_sh.at[pl.ds(base, RPT)],
                        ds_out.at[c, pl.ds(base, RPT)])
        pltpu.sync_copy(dr_sh.at[pl.ds(base, RPT)],
                        dr_out.at[c, pl.ds(base, RPT)])

    return deg_kernel(s_p, r_p)


def _scale_rows(dst, src, sbuf, nlanes):
    @plsc.parallel_loop(0, GPT, unroll=4)
    def _(g):
        row0 = g * 8
        for k in range(8):
            sv = sbuf[g, pl.ds(16 * k, 16)]
            for h in range(0, nlanes, 16):
                dst[row0 + k, pl.ds(h, 16)] = \
                    src[row0 + k, pl.ds(h, 16)] * sv


def _sc_gcn1(h, is1_pk, ir1_pk, s_p, r_p):
    d = HG

    @functools.partial(
        pl.kernel, mesh=_mesh(),
        out_type=jax.ShapeDtypeStruct((2, NP, d), jnp.float32),
        compiler_params=_sc_params,
        scratch_types=[
            pltpu.VMEM((CH, LW), jnp.int32),
            pltpu.VMEM((CH, LW), jnp.int32),
            pltpu.VMEM((4, LW, d), jnp.float32),
            pltpu.VMEM((RPT, d), jnp.float32),
            pltpu.VMEM((GPT, 128), jnp.float32),
            pltpu.VMEM_SHARED((NP, d), jnp.float32),
            pltpu.VMEM_SHARED((NP, d), jnp.float32),
        ] + [pltpu.SemaphoreType.DMA] * 8)
    def k1(h_hbm, is1_hbm, ir1_hbm, s_hbm, r_hbm, out_hbm, sidx_v, ridx_v,
           rows_v, nbuf, sbuf, acc_sh, tab_sh, gs0, gs1, gs2, gs3, ss0, ss1,
           ss2, ss3):
        c = lax.axis_index("c")
        s = lax.axis_index("s")
        wid = s * 2 + c
        base = s * RPT
        pltpu.sync_copy(s_hbm.at[wid], sidx_v)
        pltpu.sync_copy(r_hbm.at[wid], ridx_v)
        pltpu.sync_copy(h_hbm.at[pl.ds(base, RPT)], nbuf)
        pltpu.sync_copy(is1_hbm.at[pl.ds(s * GPT, GPT)], sbuf)
        _scale_rows(nbuf, nbuf, sbuf, d)
        pltpu.sync_copy(nbuf, tab_sh.at[pl.ds(base, RPT)])

        @pl.when(c == 0)
        def _():
            pltpu.sync_copy(nbuf, acc_sh.at[pl.ds(base, RPT)])

        @pl.when(c == 1)
        def _():
            @plsc.parallel_loop(0, LW, unroll=4)
            def _(i):
                for l in range(0, d, 16):
                    rows_v[0, i, pl.ds(l, 16)] = jnp.zeros((16,),
                                                           jnp.float32)

            for off, sz in _SLABS:
                pltpu.sync_copy(rows_v.at[0, pl.ds(0, sz)],
                                acc_sh.at[pl.ds(base + off, sz)])

        plsc.subcore_barrier()

        gsems = (gs0, gs1, gs2, gs3)
        ssems = (ss0, ss1, ss2, ss3)
        for b in range(4):
            pltpu.async_copy(tab_sh.at[sidx_v.at[b]], rows_v.at[b], gsems[b])

        @pl.loop(0, CH // 4)
        def _(jj):
            i0 = jj * 4
            for b in range(4):
                i = i0 + b
                pltpu.make_async_copy(tab_sh.at[sidx_v.at[i]], rows_v.at[b],
                                      gsems[b]).wait()
                pltpu.async_copy(rows_v.at[b], acc_sh.at[ridx_v.at[i]],
                                 ssems[b], add=True)
                pb = (b + 3) % 4

                @pl.when((i >= 1) & (i < CH - 3))
                def _():
                    pltpu.make_async_copy(rows_v.at[pb],
                                          acc_sh.at[ridx_v.at[i - 1]],
                                          ssems[pb]).wait()
                    pltpu.async_copy(tab_sh.at[sidx_v.at[i + 3]],
                                     rows_v.at[pb], gsems[pb])

        for b in range(4):
            i = CH - 4 + b
            pltpu.make_async_copy(rows_v.at[b], acc_sh.at[ridx_v.at[i]],
                                  ssems[b]).wait()
        plsc.subcore_barrier()

        pltpu.sync_copy(acc_sh.at[pl.ds(base, RPT)], nbuf)
        pltpu.sync_copy(ir1_hbm.at[pl.ds(s * GPT, GPT)], sbuf)
        _scale_rows(nbuf, nbuf, sbuf, d)
        pltpu.sync_copy(nbuf, out_hbm.at[c, pl.ds(base, RPT)])

    return k1(h, is1_pk, ir1_pk, s_p, r_p)


def _sc_gcn2(zh3, is2_pk, ir2_pk, s_p, r_p):
    d = 48
    HR0, HR1 = 320, RPT - 320
    HG0, HG1 = HR0 // 8, HR1 // 8

    @functools.partial(
        pl.kernel, mesh=_mesh(),
        out_type=jax.ShapeDtypeStruct((2, NP, d), jnp.float32),
        compiler_params=_sc_params,
        scratch_types=[
            pltpu.VMEM((CH, LW), jnp.int32),
            pltpu.VMEM((CH, LW), jnp.int32),
            pltpu.VMEM((4, LW, d), jnp.float32),
            pltpu.VMEM((HR0, HG), jnp.float32),
            pltpu.VMEM((HR0, d), jnp.float32),
            pltpu.VMEM((HG0, 128), jnp.float32),
            pltpu.VMEM_SHARED((NP, d), jnp.float32),
            pltpu.VMEM_SHARED((NP, d), jnp.float32),
        ] + [pltpu.SemaphoreType.DMA] * 8)
    def k2(z_hbm, is2_hbm, ir2_hbm, s_hbm, r_hbm, out_hbm, sidx_v, ridx_v,
           rows_v, zbuf, ubuf, sbuf, acc_sh, tab_sh, gs0, gs1, gs2, gs3,
           ss0, ss1, ss2, ss3):
        c = lax.axis_index("c")
        s = lax.axis_index("s")
        wid = s * 2 + c
        base = s * RPT
        pltpu.sync_copy(s_hbm.at[wid], sidx_v)
        pltpu.sync_copy(r_hbm.at[wid], ridx_v)

        if True:
            for p, (hoff, hr, hgrp) in enumerate(((0, HR0, HG0),
                                                  (HR0, HR1, HG1))):
                b0 = base + hoff
                if p == 0:
                    pltpu.sync_copy(z_hbm.at[pl.ds(b0, hr)],
                                    zbuf.at[pl.ds(0, hr)])
                else:
                    @pl.when(s < 15)
                    def _():
                        pltpu.sync_copy(z_hbm.at[pl.ds(b0, hr)],
                                        zbuf.at[pl.ds(0, hr)])

                    @pl.when(s == 15)
                    def _():
                        pltpu.sync_copy(
                            z_hbm.at[pl.ds(b0, NN - 15 * RPT - HR0)],
                            zbuf.at[pl.ds(0, NN - 15 * RPT - HR0)])
                pltpu.sync_copy(
                    is2_hbm.at[pl.ds(s * GPT + hoff // 8, hgrp)],
                    sbuf.at[pl.ds(0, hgrp)])

                @plsc.parallel_loop(0, hgrp, unroll=4)
                def _(g):
                    row0 = g * 8
                    for k in range(8):
                        sv = sbuf[g, pl.ds(16 * k, 16)]
                        for hh in range(0, HG, 16):
                            ubuf[row0 + k, pl.ds(hh, 16)] = \
                                zbuf[row0 + k, pl.ds(hh, 16)] * sv
                        ubuf[row0 + k, pl.ds(HG, 16)] = sv

                pltpu.sync_copy(ubuf.at[pl.ds(0, hr)],
                                tab_sh.at[pl.ds(b0, hr)])

            @plsc.parallel_loop(0, LW, unroll=4)
            def _(i):
                for l in range(0, d, 16):
                    rows_v[0, i, pl.ds(l, 16)] = jnp.zeros((16,),
                                                           jnp.float32)

            for off, sz in _SLABS:
                pltpu.sync_copy(rows_v.at[0, pl.ds(0, sz)],
                                acc_sh.at[pl.ds(base + off, sz)])
            plsc.subcore_barrier()

            gsems = (gs0, gs1, gs2, gs3)
            ssems = (ss0, ss1, ss2, ss3)
            for b in range(4):
                pltpu.async_copy(tab_sh.at[sidx_v.at[b]], rows_v.at[b],
                                 gsems[b])

            @pl.loop(0, CH // 4)
            def _(jj):
                i0 = jj * 4
                for b in range(4):
                    i = i0 + b
                    pltpu.make_async_copy(tab_sh.at[sidx_v.at[i]],
                                          rows_v.at[b], gsems[b]).wait()
                    pltpu.async_copy(rows_v.at[b], acc_sh.at[ridx_v.at[i]],
                                     ssems[b], add=True)
                    pb = (b + 3) % 4

                    @pl.when((i >= 1) & (i < CH - 3))
                    def _():
                        pltpu.make_async_copy(rows_v.at[pb],
                                              acc_sh.at[ridx_v.at[i - 1]],
                                              ssems[pb]).wait()
                        pltpu.async_copy(tab_sh.at[sidx_v.at[i + 3]],
                                         rows_v.at[pb], gsems[pb])

            for b in range(4):
                i = CH - 4 + b
                pltpu.make_async_copy(rows_v.at[b], acc_sh.at[ridx_v.at[i]],
                                      ssems[b]).wait()
            plsc.subcore_barrier()

            for p, (hoff, hr, hgrp) in enumerate(((0, HR0, HG0),
                                                  (HR0, HR1, HG1))):
                b0 = base + hoff
                pltpu.sync_copy(acc_sh.at[pl.ds(b0, hr)],
                                ubuf.at[pl.ds(0, hr)])
                pltpu.sync_copy(
                    ir2_hbm.at[pl.ds(s * GPT + hoff // 8, hgrp)],
                    sbuf.at[pl.ds(0, hgrp)])

                @plsc.parallel_loop(0, hgrp, unroll=4)
                def _(g):
                    row0 = g * 8
                    for k in range(8):
                        sv = sbuf[g, pl.ds(16 * k, 16)]
                        for hh in range(0, d, 16):
                            ubuf[row0 + k, pl.ds(hh, 16)] = \
                                ubuf[row0 + k, pl.ds(hh, 16)] * sv

                pltpu.sync_copy(ubuf.at[pl.ds(0, hr)],
                                out_hbm.at[c, pl.ds(b0, hr)])

    return k2(zh3, is2_pk, ir2_pk, s_p, r_p)


def _row_spec(width, rows=None):
    return pl.BlockSpec(((rows or RB), width), lambda i: (i, 0))


def _pair_spec(width, rows=None):
    return pl.BlockSpec((2, (rows or RB), width), lambda i: (0, i, 0))


def _full_spec(shape):
    return pl.BlockSpec(shape, lambda i: tuple(0 for _ in shape))


def _tc_h(nodes_p, w_enc, b_enc):
    def body(n_ref, w_ref, b_ref, h_ref):
        h = jnp.dot(n_ref[...], w_ref[...],
                    preferred_element_type=jnp.float32,
                    precision=lax.Precision.HIGHEST)
        h_ref[...] = jnp.maximum(h + b_ref[...], 0.0)

    return pl.pallas_call(
        body,
        grid=(GB,),
        in_specs=[_row_spec(DFEAT), _full_spec((DFEAT, HG)),
                  _full_spec((1, HG))],
        out_specs=_row_spec(HG),
        out_shape=jax.ShapeDtypeStruct((NP, HG), jnp.float32),
    )(nodes_p, w_enc, b_enc)


def _tc_scales(ds_pk, dr_pk):
    def body(ds_ref, dr_ref, is1_ref, ir1_ref, is2_ref, ir2_ref):
        ds = ds_ref[0] + ds_ref[1]
        dr = dr_ref[0] + dr_ref[1]
        is1_ref[...] = lax.rsqrt(ds + 1.0)
        ir1_ref[...] = lax.rsqrt(dr + 1.0)
        is2_ref[...] = lax.rsqrt(jnp.maximum(ds, 1.0))
        ir2_ref[...] = lax.rsqrt(jnp.maximum(dr, 1.0))

    hb = NPK // 2
    out = jax.ShapeDtypeStruct((NPK, 128), jnp.float32)
    return pl.pallas_call(
        body,
        grid=(2,),
        in_specs=[_pair_spec(128, hb), _pair_spec(128, hb)],
        out_specs=tuple(_row_spec(128, hb) for _ in range(4)),
        out_shape=(out, out, out, out),
    )(ds_pk, dr_pk)


def _tc_middle(q, w_fc, b_fc, w_mean, b_mean, w_logstd, b_logstd, eps,
               w_dec, b_dec):
    def body(q_ref, wfc_ref, bfc_ref, wm_ref, bm_ref, wl_ref, bl_ref,
             eps_ref, wd_ref, bd_ref, mean_ref, logstd_ref, zh_ref):
        x = q_ref[0] + q_ref[1]
        x = jnp.dot(x, wfc_ref[...], preferred_element_type=jnp.float32,
                    precision=lax.Precision.HIGHEST)
        x = jnp.maximum(x + bfc_ref[...], 0.0)
        mean = jnp.dot(x, wm_ref[...], preferred_element_type=jnp.float32,
                       precision=lax.Precision.HIGHEST) + bm_ref[...]
        logstd = jnp.dot(x, wl_ref[...], preferred_element_type=jnp.float32,
                         precision=lax.Precision.HIGHEST) + bl_ref[...]
        z = mean + jnp.exp(logstd) * eps_ref[...]
        zh = jnp.dot(z, wd_ref[...], preferred_element_type=jnp.float32,
                     precision=lax.Precision.HIGHEST)
        zh_ref[...] = jnp.maximum(zh + bd_ref[...], 0.0)
        mean_ref[...] = mean
        logstd_ref[...] = logstd

    return pl.pallas_call(
        body,
        out_shape=(jax.ShapeDtypeStruct((B, LAT), jnp.float32),
                   jax.ShapeDtypeStruct((B, LAT), jnp.float32),
                   jax.ShapeDtypeStruct((B, N * HG), jnp.float32)),
    )(q, w_fc, b_fc, w_mean, b_mean, w_logstd, b_logstd, eps, w_dec, b_dec)


def _tc_final(q, w_out, b_out):
    def body(q_ref, w_ref, b_ref, o_ref):
        t = q_ref[0] + q_ref[1]
        u = lax.slice(t, (0, 0), (1000, HG))
        v = lax.slice(t, (0, HG), (1000, HG + 1))
        o_ref[...] = jnp.dot(u, w_ref[...],
                             preferred_element_type=jnp.float32,
                             precision=lax.Precision.HIGHEST) + v * b_ref[...]

    return pl.pallas_call(
        body,
        grid=(10,),
        in_specs=[_pair_spec(48, 1000), _full_spec((HG, OUT)),
                  _full_spec((1, OUT))],
        out_specs=_row_spec(OUT, 1000),
        out_shape=jax.ShapeDtypeStruct((NN, OUT), jnp.float32),
    )(q, w_out, b_out)


def kernel(nodes, senders, receivers, eps, w_enc, b_enc, w_fc, b_fc, w_mean,
           b_mean, w_logstd, b_logstd, w_dec, b_dec, w_out, b_out):
    padv = NN + (jnp.arange(PADW, dtype=jnp.int32) % (NP - NN))
    padb = jnp.broadcast_to(padv[None, :], (NW, PADW))
    s_p = jnp.concatenate(
        [senders.reshape(NW, EWR), padb], axis=1).reshape(NW, CH, LW)
    r_p = jnp.concatenate(
        [receivers.reshape(NW, EWR), padb], axis=1).reshape(NW, CH, LW)

    degs_p, degr_p = _sc_degrees(s_p, r_p)

    nodes_p = jnp.pad(nodes, ((0, NP - NN), (0, 0)))
    h = _tc_h(nodes_p, w_enc, b_enc.reshape(1, HG))

    is1, ir1, is2, ir2 = _tc_scales(degs_p.reshape(2, NPK, 128),
                                    degr_p.reshape(2, NPK, 128))

    seg1 = _sc_gcn1(h, is1, ir1, s_p, r_p)

    q1 = seg1.reshape(2, NP * HG // 128, 128)[:, :NN * HG // 128]
    q1 = q1.reshape(2, B, N * HG)
    mean, log_std, zh = _tc_middle(
        q1, w_fc, b_fc.reshape(1, HFC), w_mean, b_mean.reshape(1, LAT),
        w_logstd, b_logstd.reshape(1, LAT), eps, w_dec,
        b_dec.reshape(1, N * HG))

    seg2 = _sc_gcn2(zh.reshape(NN, HG), is2, ir2, s_p, r_p)

    outp = _tc_final(seg2, w_out, b_out.reshape(1, OUT))
    return mean, log_std, outp
